# Initial kernel scaffold; baseline (speedup 1.0000x reference)
import jax
import jax.numpy as jnp
from jax.experimental import pallas as pl


def kernel(x_employee, x_shift, x_variable, x_constraint, edge_same_day, edge_var_emp, edge_var_shift, W_proj_emp, b_proj_emp, W_proj_shift, b_proj_shift, W_proj_var, b_proj_var, W_proj_con, b_proj_con, W_gat, att_src, att_dst, b_gat, W_inj_emp, b_inj_emp, W_inj_shift, b_inj_shift, Wq_emp, Wk_emp, Wq_shift, Wk_shift, W_fuse, b_fuse):
    raise NotImplementedError("write your pallas kernel here")


# speedup vs baseline: 3.3808x; 3.3808x over previous
import functools

import jax
import jax.numpy as jnp
from jax import lax
from jax.experimental import pallas as pl
from jax.experimental.pallas import tpu as pltpu
from jax.experimental.pallas import tpu_sc as plsc

HID = 64
HEADS = 4
SCALE = 8.0
NC = 2
NS = 16
NW = NC * NS
C = 80

N_VAR = 100000
N_SHIFT = 10000
N_EMP = 10000
E_SD = 320000
E_VI = 640000
DEN_G = 40960
DEN_I = 100352
RNG = 25000

f32 = jnp.float32
i32 = jnp.int32


def _ent_pre(x, W, b, Wk, Wv, bv):
  N = x.shape[0]
  BR = 1000
  def body(x_ref, W_ref, b_ref, Wk_ref, Wv_ref, bv_ref, h_ref, k_ref, v_ref):
    h = jnp.maximum(x_ref[...] @ W_ref[...] + b_ref[...], 0.0)
    h_ref[...] = h
    k_ref[...] = h @ Wk_ref[...]
    v_ref[...] = h @ Wv_ref[...] + bv_ref[...]
  K = x.shape[1]
  return pl.pallas_call(
      body,
      grid=(N // BR,),
      in_specs=[
          pl.BlockSpec((BR, K), lambda i: (i, 0)),
          pl.BlockSpec((K, HID), lambda i: (0, 0)),
          pl.BlockSpec((1, HID), lambda i: (0, 0)),
          pl.BlockSpec((HID, HID), lambda i: (0, 0)),
          pl.BlockSpec((HID, HID), lambda i: (0, 0)),
          pl.BlockSpec((1, HID), lambda i: (0, 0)),
      ],
      out_specs=[
          pl.BlockSpec((BR, HID), lambda i: (i, 0)),
          pl.BlockSpec((BR, HID), lambda i: (i, 0)),
          pl.BlockSpec((BR, HID), lambda i: (i, 0)),
      ],
      out_shape=[jax.ShapeDtypeStruct((N, HID), f32)] * 3,
  )(x, W, b.reshape(1, HID), Wk, Wv, bv.reshape(1, HID))


def _shift_pre(x, W, b, W_gat, A_src, A_dst):
  N = x.shape[0]
  BR = 1000
  K = x.shape[1]
  def body(x_ref, W_ref, b_ref, Wg_ref, As_ref, Ad_ref,
           h_ref, hg_ref, as_ref, ad_ref):
    h = jnp.maximum(x_ref[...] @ W_ref[...] + b_ref[...], 0.0)
    hg = h @ Wg_ref[...]
    h_ref[...] = h
    hg_ref[...] = hg
    as_ref[...] = hg @ As_ref[...]
    ad_ref[...] = hg @ Ad_ref[...]
  return pl.pallas_call(
      body,
      grid=(N // BR,),
      in_specs=[
          pl.BlockSpec((BR, K), lambda i: (i, 0)),
          pl.BlockSpec((K, HID), lambda i: (0, 0)),
          pl.BlockSpec((1, HID), lambda i: (0, 0)),
          pl.BlockSpec((HID, HEADS * HID), lambda i: (0, 0)),
          pl.BlockSpec((HEADS * HID, HEADS), lambda i: (0, 0)),
          pl.BlockSpec((HEADS * HID, HEADS), lambda i: (0, 0)),
      ],
      out_specs=[
          pl.BlockSpec((BR, HID), lambda i: (i, 0)),
          pl.BlockSpec((BR, HEADS * HID), lambda i: (i, 0)),
          pl.BlockSpec((BR, HEADS), lambda i: (i, 0)),
          pl.BlockSpec((BR, HEADS), lambda i: (i, 0)),
      ],
      out_shape=[
          jax.ShapeDtypeStruct((N, HID), f32),
          jax.ShapeDtypeStruct((N, HEADS * HID), f32),
          jax.ShapeDtypeStruct((N, HEADS), f32),
          jax.ShapeDtypeStruct((N, HEADS), f32),
      ],
  )(x, W, b.reshape(1, HID), W_gat, A_src, A_dst)


def _var_pre(x, W, b, Wq1, Wq2):
  N = x.shape[0]
  BR = 1000
  K = x.shape[1]
  def body(x_ref, W_ref, b_ref, W1_ref, W2_ref, h_ref, q1_ref, q2_ref):
    h = jnp.maximum(x_ref[...] @ W_ref[...] + b_ref[...], 0.0)
    h_ref[...] = h
    q1_ref[...] = h @ W1_ref[...]
    q2_ref[...] = h @ W2_ref[...]
  return pl.pallas_call(
      body,
      grid=(N // BR,),
      in_specs=[
          pl.BlockSpec((BR, K), lambda i: (i, 0)),
          pl.BlockSpec((K, HID), lambda i: (0, 0)),
          pl.BlockSpec((1, HID), lambda i: (0, 0)),
          pl.BlockSpec((HID, HID), lambda i: (0, 0)),
          pl.BlockSpec((HID, HID), lambda i: (0, 0)),
      ],
      out_specs=[
          pl.BlockSpec((BR, HID), lambda i: (i, 0)),
          pl.BlockSpec((BR, HID), lambda i: (i, 0)),
          pl.BlockSpec((BR, HID), lambda i: (i, 0)),
      ],
      out_shape=[jax.ShapeDtypeStruct((N, HID), f32)] * 3,
  )(x, W, b.reshape(1, HID), Wq1, Wq2)


def _denr(d0, d1, clip):
  M = d0.shape[0]
  rows = M // 128
  def body(a_ref, b_ref, o_ref):
    o_ref[...] = 1.0 / jnp.maximum(a_ref[...] + b_ref[...], clip)
  out = pl.pallas_call(
      body,
      out_shape=jax.ShapeDtypeStruct((rows, 128), f32),
  )(d0.reshape(rows, 128), d1.reshape(rows, 128))
  return out.reshape(M)


def _post_gat(agg0, agg1, b_gat, h0, Wk, Wv, bv):
  N = h0.shape[0]
  BR = 1000
  def body(a0_ref, a1_ref, bg_ref, h0_ref, Wk_ref, Wv_ref, bv_ref,
           k_ref, v_ref):
    g = a0_ref[...] + a1_ref[...] + bg_ref[...]
    hs = jnp.maximum(g, 0.0) + h0_ref[...]
    k_ref[...] = hs @ Wk_ref[...]
    v_ref[...] = hs @ Wv_ref[...] + bv_ref[...]
  return pl.pallas_call(
      body,
      grid=(N // BR,),
      in_specs=[
          pl.BlockSpec((BR, HID), lambda i: (i, 0)),
          pl.BlockSpec((BR, HID), lambda i: (i, 0)),
          pl.BlockSpec((1, HID), lambda i: (0, 0)),
          pl.BlockSpec((BR, HID), lambda i: (i, 0)),
          pl.BlockSpec((HID, HID), lambda i: (0, 0)),
          pl.BlockSpec((HID, HID), lambda i: (0, 0)),
          pl.BlockSpec((1, HID), lambda i: (0, 0)),
      ],
      out_specs=[
          pl.BlockSpec((BR, HID), lambda i: (i, 0)),
          pl.BlockSpec((BR, HID), lambda i: (i, 0)),
      ],
      out_shape=[jax.ShapeDtypeStruct((N, HID), f32)] * 2,
  )(agg0, agg1, b_gat.reshape(1, HID), h0, Wk, Wv, bv.reshape(1, HID))


def _fuse(hv, mE, mS, W1, W2, W3, b):
  N = hv.shape[0]
  BR = 1000
  def body(hv_ref, mE_ref, mS_ref, W1_ref, W2_ref, W3_ref, b_ref, o_ref):
    z = (hv_ref[...] @ W1_ref[...] + mE_ref[...] @ W2_ref[...]
         + mS_ref[...] @ W3_ref[...] + b_ref[...])
    z = jnp.maximum(z, 0.0)
    o_ref[...] = jnp.sqrt(jnp.sum(z * z, axis=1, keepdims=True))
  out = pl.pallas_call(
      body,
      grid=(N // BR,),
      in_specs=[
          pl.BlockSpec((BR, HID), lambda i: (i, 0)),
          pl.BlockSpec((BR, HID), lambda i: (i, 0)),
          pl.BlockSpec((BR, HID), lambda i: (i, 0)),
          pl.BlockSpec((HID, HID), lambda i: (0, 0)),
          pl.BlockSpec((HID, HID), lambda i: (0, 0)),
          pl.BlockSpec((HID, HID), lambda i: (0, 0)),
          pl.BlockSpec((1, HID), lambda i: (0, 0)),
      ],
      out_specs=pl.BlockSpec((BR, 1), lambda i: (i, 0)),
      out_shape=jax.ShapeDtypeStruct((N, 1), f32),
  )(hv, mE, mS, W1, W2, W3, b.reshape(1, HID))
  return out.reshape(N)


def _sc_mesh():
  return plsc.VectorSubcoreMesh(core_axis_name="c", subcore_axis_name="s")


def _wids():
  cid = lax.axis_index("c")
  sid = lax.axis_index("s")
  return cid, sid, sid * NC + cid


def _gat_edge(src, dst, aS, aD, zden):
  EW = E_SD // NW
  nch = EW // C

  @functools.partial(
      pl.kernel, mesh=_sc_mesh(),
      compiler_params=pltpu.CompilerParams(needs_layout_passes=False, use_tc_tiling_on_sc=False),
      out_type=(jax.ShapeDtypeStruct((E_SD * HEADS,), f32),
                jax.ShapeDtypeStruct((NC, DEN_G), f32)),
      scratch_types=[
          pltpu.VMEM((N_SHIFT * HEADS,), f32),
          pltpu.VMEM((N_SHIFT * HEADS,), f32),
          pltpu.VMEM((C,), i32),
          pltpu.VMEM((C,), i32),
          pltpu.VMEM((C * HEADS,), f32),
          pltpu.VMEM((HEADS, C), f32),
          pltpu.VMEM((HEADS, C), i32),
          pltpu.VMEM_SHARED((DEN_G,), f32),
          pltpu.SemaphoreType.DMA,
      ])
  def body(src_hbm, dst_hbm, aS_hbm, aD_hbm, zden_hbm, ex_hbm, denP_hbm,
           aS_v, aD_v, src_c, dst_c, exc, exh, ibuf, den_sh, sem):
    cid, sid, wid = _wids()
    base = wid * EW
    pltpu.sync_copy(aS_hbm, aS_v)
    pltpu.sync_copy(aD_hbm, aD_v)

    @pl.when(sid == 0)
    def _():
      pltpu.sync_copy(zden_hbm, den_sh)
    plsc.subcore_barrier()

    def chunk(g, carry):
      eb = base + g * C
      pltpu.sync_copy(src_hbm.at[pl.ds(eb, C)], src_c)
      pltpu.sync_copy(dst_hbm.at[pl.ds(eb, C)], dst_c)
      for i in range(C // 16):
        loc16 = lax.iota(i32, 16) + i * 16
        s16 = src_c[pl.ds(i * 16, 16)]
        d16 = dst_c[pl.ds(i * 16, 16)]
        for h in range(HEADS):
          h16 = jnp.full((16,), h, i32)
          e16 = (plsc.load_gather(aS_v, [s16 * HEADS + h16])
                 + plsc.load_gather(aD_v, [d16 * HEADS + h16]))
          e16 = jnp.where(e16 >= 0.0, e16, 0.2 * e16)
          ex16 = jnp.exp(e16)
          plsc.store_scatter(exc, [loc16 * HEADS + h16], ex16)
          exh[h, pl.ds(i * 16, 16)] = ex16
          ibuf[h, pl.ds(i * 16, 16)] = d16 * HEADS + h16
      pltpu.sync_copy(exc, ex_hbm.at[pl.ds(eb * HEADS, C * HEADS)])
      for h in range(HEADS):
        pltpu.sync_copy(exh.at[h], den_sh.at[ibuf.at[h]], add=True)
      return carry

    lax.fori_loop(0, nch, chunk, 0)
    plsc.subcore_barrier()

    @pl.when(sid == 0)
    def _():
      pltpu.sync_copy(den_sh, denP_hbm.at[cid])

  return body(src, dst, aS, aD, zden)


def _gat_agg(src, dst, ex, denr, hg, zagg):
  EW = E_SD // NW
  nch = EW // C

  @functools.partial(
      pl.kernel, mesh=_sc_mesh(),
      compiler_params=pltpu.CompilerParams(needs_layout_passes=False, use_tc_tiling_on_sc=False),
      out_type=jax.ShapeDtypeStruct((NC, N_SHIFT, HID), f32),
      scratch_types=[
          pltpu.VMEM((DEN_G,), f32),
          pltpu.VMEM((C,), i32),
          pltpu.VMEM((C,), i32),
          pltpu.VMEM((C * HEADS,), f32),
          pltpu.VMEM((C, HEADS * HID), f32),
          pltpu.VMEM((C, HID), f32),
          pltpu.VMEM_SHARED((N_SHIFT, HID), f32),
          pltpu.SemaphoreType.DMA,
      ])
  def body(src_hbm, dst_hbm, ex_hbm, denr_hbm, hg_hbm, zagg_hbm, aggP_hbm,
           denr_v, src_c, dst_c, exc, rows, cvals, agg_sh, sem):
    cid, sid, wid = _wids()
    base = wid * EW
    pltpu.sync_copy(denr_hbm, denr_v)

    @pl.when(sid == 0)
    def _():
      pltpu.sync_copy(zagg_hbm, agg_sh)
    plsc.subcore_barrier()

    def chunk(g, carry):
      eb = base + g * C
      pltpu.sync_copy(src_hbm.at[pl.ds(eb, C)], src_c)
      pltpu.sync_copy(dst_hbm.at[pl.ds(eb, C)], dst_c)
      pltpu.sync_copy(ex_hbm.at[pl.ds(eb * HEADS, C * HEADS)], exc)
      pltpu.async_copy(hg_hbm.at[src_c], rows, sem).wait()
      for i in range(C // 16):
        e16 = lax.iota(i32, 16) + i * 16
        d16 = dst_c[pl.ds(i * 16, 16)]
        alphas = []
        for h in range(HEADS):
          h16 = jnp.full((16,), h, i32)
          exv = plsc.load_gather(exc, [e16 * HEADS + h16])
          drv = plsc.load_gather(denr_v, [d16 * HEADS + h16])
          alphas.append(exv * drv * 0.25)

        def colbody(ccol, carry2):
          c16 = jnp.zeros((16,), i32) + ccol
          acc = jnp.zeros((16,), f32)
          for h in range(HEADS):
            acc = acc + alphas[h] * plsc.load_gather(
                rows, [e16, c16 + h * HID])
          plsc.store_scatter(cvals, [e16, c16], acc)
          return carry2

        lax.fori_loop(0, HID, colbody, 0)
      pltpu.sync_copy(cvals, agg_sh.at[dst_c], add=True)
      return carry

    lax.fori_loop(0, nch, chunk, 0)
    plsc.subcore_barrier()

    @pl.when(sid == 0)
    def _():
      pltpu.sync_copy(agg_sh, aggP_hbm.at[cid])

  return body(src, dst, ex, denr, hg, zagg)


def _inj_score(dst, src, q, k, zden):
  EW = E_VI // NW
  nch = EW // C

  @functools.partial(
      pl.kernel, mesh=_sc_mesh(),
      compiler_params=pltpu.CompilerParams(needs_layout_passes=False, use_tc_tiling_on_sc=False),
      out_type=(jax.ShapeDtypeStruct((E_VI,), f32),
                jax.ShapeDtypeStruct((NC, DEN_I), f32)),
      scratch_types=[
          pltpu.VMEM((C,), i32),
          pltpu.VMEM((C,), i32),
          pltpu.VMEM((C, HID), f32),
          pltpu.VMEM((C, HID), f32),
          pltpu.VMEM((C,), f32),
          pltpu.VMEM_SHARED((DEN_I,), f32),
          pltpu.SemaphoreType.DMA,
      ])
  def body(dst_hbm, src_hbm, q_hbm, k_hbm, zden_hbm, ex_hbm, denP_hbm,
           dst_c, src_c, qrows, krows, sbuf, den_sh, sem):
    cid, sid, wid = _wids()
    base = wid * EW

    @pl.when(sid == 0)
    def _():
      pltpu.sync_copy(zden_hbm, den_sh)
    plsc.subcore_barrier()

    def chunk(g, carry):
      eb = base + g * C
      pltpu.sync_copy(dst_hbm.at[pl.ds(eb, C)], dst_c)
      pltpu.sync_copy(src_hbm.at[pl.ds(eb, C)], src_c)
      pltpu.async_copy(q_hbm.at[dst_c], qrows, sem).wait()
      pltpu.async_copy(k_hbm.at[src_c], krows, sem).wait()
      for i in range(C // 16):
        e16 = lax.iota(i32, 16) + i * 16

        def colbody(ccol, acc):
          c16 = jnp.zeros((16,), i32) + ccol
          return acc + (plsc.load_gather(qrows, [e16, c16])
                        * plsc.load_gather(krows, [e16, c16]))

        acc = lax.fori_loop(0, HID, colbody, jnp.zeros((16,), f32))
        sbuf[pl.ds(i * 16, 16)] = jnp.exp(acc * (1.0 / SCALE))
      pltpu.sync_copy(sbuf, ex_hbm.at[pl.ds(eb, C)])
      pltpu.sync_copy(sbuf, den_sh.at[dst_c], add=True)
      return carry

    lax.fori_loop(0, nch, chunk, 0)
    plsc.subcore_barrier()

    @pl.when(sid == 0)
    def _():
      pltpu.sync_copy(den_sh, denP_hbm.at[cid])

  return body(dst, src, q, k, zden)


def _alphaize(dst, ex, denr):
  EW = E_VI // NW
  nch = EW // C

  @functools.partial(
      pl.kernel, mesh=_sc_mesh(),
      compiler_params=pltpu.CompilerParams(needs_layout_passes=False, use_tc_tiling_on_sc=False),
      out_type=jax.ShapeDtypeStruct((E_VI,), f32),
      scratch_types=[
          pltpu.VMEM((DEN_I,), f32),
          pltpu.VMEM((C,), i32),
          pltpu.VMEM((C,), f32),
          pltpu.SemaphoreType.DMA,
      ])
  def body(dst_hbm, ex_hbm, denr_hbm, al_hbm, denr_v, dst_c, buf, sem):
    cid, sid, wid = _wids()
    base = wid * EW
    pltpu.sync_copy(denr_hbm, denr_v)

    def chunk(g, carry):
      eb = base + g * C
      pltpu.sync_copy(dst_hbm.at[pl.ds(eb, C)], dst_c)
      pltpu.sync_copy(ex_hbm.at[pl.ds(eb, C)], buf)
      for i in range(C // 16):
        sl = pl.ds(i * 16, 16)
        buf[sl] = buf[sl] * plsc.load_gather(denr_v, [dst_c[sl]])
      pltpu.sync_copy(buf, al_hbm.at[pl.ds(eb, C)])
      return carry

    lax.fori_loop(0, nch, chunk, 0)

  return body(dst, ex, denr)


def _inj_agg(dst, src, alpha, vals, zmsg):
  EW = E_VI // NS
  nch = EW // C

  @functools.partial(
      pl.kernel, mesh=_sc_mesh(),
      compiler_params=pltpu.CompilerParams(needs_layout_passes=False, use_tc_tiling_on_sc=False),
      out_type=jax.ShapeDtypeStruct((N_VAR, HID), f32),
      scratch_types=[
          pltpu.VMEM((C,), i32),
          pltpu.VMEM((C,), i32),
          pltpu.VMEM((C,), f32),
          pltpu.VMEM((C,), f32),
          pltpu.VMEM((C,), i32),
          pltpu.VMEM((C, HID), f32),
          pltpu.VMEM((C, HID), f32),
          pltpu.VMEM_SHARED((RNG, HID), f32),
          pltpu.SemaphoreType.DMA,
      ])
  def body(dst_hbm, src_hbm, al_hbm, vals_hbm, zmsg_hbm, msg_hbm,
           dst_c, src_c, exb, ab, ib, rows, obuf, msg_sh, sem):
    cid, sid, wid = _wids()
    base = sid * EW
    for r in range(2):
      rb = (lax.axis_index("c") * 2 + r) * RNG

      @pl.when(sid == 0)
      def _():
        pltpu.sync_copy(zmsg_hbm, msg_sh)
      plsc.subcore_barrier()

      def chunk(g, carry):
        eb = base + g * C
        pltpu.sync_copy(dst_hbm.at[pl.ds(eb, C)], dst_c)
        pltpu.sync_copy(src_hbm.at[pl.ds(eb, C)], src_c)
        pltpu.sync_copy(al_hbm.at[pl.ds(eb, C)], exb)
        pltpu.async_copy(vals_hbm.at[src_c], rows, sem).wait()
        for i in range(C // 16):
          sl = pl.ds(i * 16, 16)
          d16 = dst_c[sl]
          inr = (d16 >= rb) & (d16 < rb + RNG)
          dloc = jnp.where(inr, d16 - rb, 0)
          ab[sl] = jnp.where(inr, exb[sl], 0.0)
          ib[sl] = dloc

        def colbody(ccol, carry2):
          c16 = jnp.zeros((16,), i32) + ccol
          for i in range(C // 16):
            e16 = lax.iota(i32, 16) + i * 16
            v16 = plsc.load_gather(rows, [e16, c16])
            plsc.store_scatter(obuf, [e16, c16],
                               v16 * ab[pl.ds(i * 16, 16)])
          return carry2

        lax.fori_loop(0, HID, colbody, 0)
        pltpu.sync_copy(obuf, msg_sh.at[ib], add=True)
        return carry

      lax.fori_loop(0, nch, chunk, 0)
      plsc.subcore_barrier()

      @pl.when(sid == 0)
      def _():
        pltpu.sync_copy(msg_sh, msg_hbm.at[pl.ds(rb, RNG)])
      plsc.subcore_barrier()

  return body(dst, src, alpha, vals, zmsg)


def kernel(x_employee, x_shift, x_variable, x_constraint, edge_same_day,
           edge_var_emp, edge_var_shift, W_proj_emp, b_proj_emp,
           W_proj_shift, b_proj_shift, W_proj_var, b_proj_var, W_proj_con,
           b_proj_con, W_gat, att_src, att_dst, b_gat, W_inj_emp, b_inj_emp,
           W_inj_shift, b_inj_shift, Wq_emp, Wk_emp, Wq_shift, Wk_shift,
           W_fuse, b_fuse):
  del x_constraint, W_proj_con, b_proj_con

  A_src = jnp.zeros((HEADS * HID, HEADS), f32)
  A_dst = jnp.zeros((HEADS * HID, HEADS), f32)
  for h in range(HEADS):
    A_src = A_src.at[h * HID:(h + 1) * HID, h].set(att_src[h])
    A_dst = A_dst.at[h * HID:(h + 1) * HID, h].set(att_dst[h])

  h_emp, k_emp, v_emp = _ent_pre(x_employee, W_proj_emp, b_proj_emp,
                                 Wk_emp, W_inj_emp, b_inj_emp)
  del h_emp
  h_shift0, hg, aS, aD = _shift_pre(x_shift, W_proj_shift, b_proj_shift,
                                    W_gat, A_src, A_dst)
  h_var, q_emp, q_shift = _var_pre(x_variable, W_proj_var, b_proj_var,
                                   Wq_emp, Wq_shift)

  src_sd = edge_same_day[0]
  dst_sd = edge_same_day[1]
  ex_g, denP_g = _gat_edge(src_sd, dst_sd, aS.reshape(-1), aD.reshape(-1),
                           jnp.zeros((DEN_G,), f32))
  denr_g = _denr(denP_g[0], denP_g[1], 1e-16)
  aggP = _gat_agg(src_sd, dst_sd, ex_g, denr_g, hg,
                  jnp.zeros((N_SHIFT, HID), f32))
  k_shift, v_shift = _post_gat(aggP[0], aggP[1], b_gat, h_shift0,
                               Wk_shift, W_inj_shift, b_inj_shift)

  zden_i = jnp.zeros((DEN_I,), f32)
  zmsg = jnp.zeros((RNG, HID), f32)

  d_ve = edge_var_emp[0]
  s_ve = edge_var_emp[1]
  ex_e, denP_e = _inj_score(d_ve, s_ve, q_emp, k_emp, zden_i)
  denr_e = _denr(denP_e[0], denP_e[1], 1e-9)
  al_e = _alphaize(d_ve, ex_e, denr_e)
  msg_e = _inj_agg(d_ve, s_ve, al_e, v_emp, zmsg)

  d_vs = edge_var_shift[0]
  s_vs = edge_var_shift[1]
  ex_s, denP_s = _inj_score(d_vs, s_vs, q_shift, k_shift, zden_i)
  denr_s = _denr(denP_s[0], denP_s[1], 1e-9)
  al_s = _alphaize(d_vs, ex_s, denr_s)
  msg_s = _inj_agg(d_vs, s_vs, al_s, v_shift, zmsg)

  W1 = W_fuse[:HID]
  W2 = W_fuse[HID:2 * HID]
  W3 = W_fuse[2 * HID:]
  return _fuse(h_var, msg_e, msg_s, W1, W2, W3, b_fuse)


# speedup vs baseline: 3.8130x; 1.1279x over previous
import functools

import jax
import jax.numpy as jnp
from jax import lax
from jax.experimental import pallas as pl
from jax.experimental.pallas import tpu as pltpu
from jax.experimental.pallas import tpu_sc as plsc

HID = 64
HEADS = 4
SCALE = 8.0
NC = 2
NS = 16
NW = NC * NS
C = 80

N_VAR = 100000
N_SHIFT = 10000
N_EMP = 10000
E_SD = 320000
E_VI = 640000
DEN_G = 40960
DEN_I = 100352
RNG = 25000

f32 = jnp.float32
i32 = jnp.int32


def _ent_pre(x, W, b, Wk, Wv, bv):
  N = x.shape[0]
  BR = 1000
  def body(x_ref, W_ref, b_ref, Wk_ref, Wv_ref, bv_ref, h_ref, k_ref, v_ref):
    h = jnp.maximum(x_ref[...] @ W_ref[...] + b_ref[...], 0.0)
    h_ref[...] = h
    k_ref[...] = h @ Wk_ref[...]
    v_ref[...] = h @ Wv_ref[...] + bv_ref[...]
  K = x.shape[1]
  return pl.pallas_call(
      body,
      grid=(N // BR,),
      in_specs=[
          pl.BlockSpec((BR, K), lambda i: (i, 0)),
          pl.BlockSpec((K, HID), lambda i: (0, 0)),
          pl.BlockSpec((1, HID), lambda i: (0, 0)),
          pl.BlockSpec((HID, HID), lambda i: (0, 0)),
          pl.BlockSpec((HID, HID), lambda i: (0, 0)),
          pl.BlockSpec((1, HID), lambda i: (0, 0)),
      ],
      out_specs=[
          pl.BlockSpec((BR, HID), lambda i: (i, 0)),
          pl.BlockSpec((BR, HID), lambda i: (i, 0)),
          pl.BlockSpec((BR, HID), lambda i: (i, 0)),
      ],
      out_shape=[jax.ShapeDtypeStruct((N, HID), f32)] * 3,
  )(x, W, b.reshape(1, HID), Wk, Wv, bv.reshape(1, HID))


def _shift_pre(x, W, b, W_gat, A_src, A_dst):
  N = x.shape[0]
  BR = 1000
  K = x.shape[1]
  def body(x_ref, W_ref, b_ref, Wg_ref, As_ref, Ad_ref,
           h_ref, hg_ref, as_ref, ad_ref):
    h = jnp.maximum(x_ref[...] @ W_ref[...] + b_ref[...], 0.0)
    hg = h @ Wg_ref[...]
    h_ref[...] = h
    hg_ref[...] = hg
    as_ref[...] = hg @ As_ref[...]
    ad_ref[...] = hg @ Ad_ref[...]
  return pl.pallas_call(
      body,
      grid=(N // BR,),
      in_specs=[
          pl.BlockSpec((BR, K), lambda i: (i, 0)),
          pl.BlockSpec((K, HID), lambda i: (0, 0)),
          pl.BlockSpec((1, HID), lambda i: (0, 0)),
          pl.BlockSpec((HID, HEADS * HID), lambda i: (0, 0)),
          pl.BlockSpec((HEADS * HID, HEADS), lambda i: (0, 0)),
          pl.BlockSpec((HEADS * HID, HEADS), lambda i: (0, 0)),
      ],
      out_specs=[
          pl.BlockSpec((BR, HID), lambda i: (i, 0)),
          pl.BlockSpec((BR, HEADS * HID), lambda i: (i, 0)),
          pl.BlockSpec((BR, HEADS), lambda i: (i, 0)),
          pl.BlockSpec((BR, HEADS), lambda i: (i, 0)),
      ],
      out_shape=[
          jax.ShapeDtypeStruct((N, HID), f32),
          jax.ShapeDtypeStruct((N, HEADS * HID), f32),
          jax.ShapeDtypeStruct((N, HEADS), f32),
          jax.ShapeDtypeStruct((N, HEADS), f32),
      ],
  )(x, W, b.reshape(1, HID), W_gat, A_src, A_dst)


def _var_pre(x, W, b, Wq1, Wq2):
  N = x.shape[0]
  BR = 1000
  K = x.shape[1]
  def body(x_ref, W_ref, b_ref, W1_ref, W2_ref, h_ref, q1_ref, q2_ref):
    h = jnp.maximum(x_ref[...] @ W_ref[...] + b_ref[...], 0.0)
    h_ref[...] = h
    q1_ref[...] = h @ W1_ref[...]
    q2_ref[...] = h @ W2_ref[...]
  return pl.pallas_call(
      body,
      grid=(N // BR,),
      in_specs=[
          pl.BlockSpec((BR, K), lambda i: (i, 0)),
          pl.BlockSpec((K, HID), lambda i: (0, 0)),
          pl.BlockSpec((1, HID), lambda i: (0, 0)),
          pl.BlockSpec((HID, HID), lambda i: (0, 0)),
          pl.BlockSpec((HID, HID), lambda i: (0, 0)),
      ],
      out_specs=[
          pl.BlockSpec((BR, HID), lambda i: (i, 0)),
          pl.BlockSpec((BR, HID), lambda i: (i, 0)),
          pl.BlockSpec((BR, HID), lambda i: (i, 0)),
      ],
      out_shape=[jax.ShapeDtypeStruct((N, HID), f32)] * 3,
  )(x, W, b.reshape(1, HID), Wq1, Wq2)


def _denr(d0, d1, clip):
  M = d0.shape[0]
  rows = M // 128
  def body(a_ref, b_ref, o_ref):
    o_ref[...] = 1.0 / jnp.maximum(a_ref[...] + b_ref[...], clip)
  out = pl.pallas_call(
      body,
      out_shape=jax.ShapeDtypeStruct((rows, 128), f32),
  )(d0.reshape(rows, 128), d1.reshape(rows, 128))
  return out.reshape(M)


def _post_gat(agg0, agg1, b_gat, h0, Wk, Wv, bv):
  N = h0.shape[0]
  BR = 1000
  def body(a0_ref, a1_ref, bg_ref, h0_ref, Wk_ref, Wv_ref, bv_ref,
           k_ref, v_ref):
    g = a0_ref[...] + a1_ref[...] + bg_ref[...]
    hs = jnp.maximum(g, 0.0) + h0_ref[...]
    k_ref[...] = hs @ Wk_ref[...]
    v_ref[...] = hs @ Wv_ref[...] + bv_ref[...]
  return pl.pallas_call(
      body,
      grid=(N // BR,),
      in_specs=[
          pl.BlockSpec((BR, HID), lambda i: (i, 0)),
          pl.BlockSpec((BR, HID), lambda i: (i, 0)),
          pl.BlockSpec((1, HID), lambda i: (0, 0)),
          pl.BlockSpec((BR, HID), lambda i: (i, 0)),
          pl.BlockSpec((HID, HID), lambda i: (0, 0)),
          pl.BlockSpec((HID, HID), lambda i: (0, 0)),
          pl.BlockSpec((1, HID), lambda i: (0, 0)),
      ],
      out_specs=[
          pl.BlockSpec((BR, HID), lambda i: (i, 0)),
          pl.BlockSpec((BR, HID), lambda i: (i, 0)),
      ],
      out_shape=[jax.ShapeDtypeStruct((N, HID), f32)] * 2,
  )(agg0, agg1, b_gat.reshape(1, HID), h0, Wk, Wv, bv.reshape(1, HID))


def _fuse(hv, mE, mS, W1, W2, W3, b):
  N = hv.shape[0]
  BR = 1000
  def body(hv_ref, mE_ref, mS_ref, W1_ref, W2_ref, W3_ref, b_ref, o_ref):
    z = (hv_ref[...] @ W1_ref[...] + mE_ref[...] @ W2_ref[...]
         + mS_ref[...] @ W3_ref[...] + b_ref[...])
    z = jnp.maximum(z, 0.0)
    o_ref[...] = jnp.sqrt(jnp.sum(z * z, axis=1, keepdims=True))
  out = pl.pallas_call(
      body,
      grid=(N // BR,),
      in_specs=[
          pl.BlockSpec((BR, HID), lambda i: (i, 0)),
          pl.BlockSpec((BR, HID), lambda i: (i, 0)),
          pl.BlockSpec((BR, HID), lambda i: (i, 0)),
          pl.BlockSpec((HID, HID), lambda i: (0, 0)),
          pl.BlockSpec((HID, HID), lambda i: (0, 0)),
          pl.BlockSpec((HID, HID), lambda i: (0, 0)),
          pl.BlockSpec((1, HID), lambda i: (0, 0)),
      ],
      out_specs=pl.BlockSpec((BR, 1), lambda i: (i, 0)),
      out_shape=jax.ShapeDtypeStruct((N, 1), f32),
  )(hv, mE, mS, W1, W2, W3, b.reshape(1, HID))
  return out.reshape(N)


def _sc_mesh():
  return plsc.VectorSubcoreMesh(core_axis_name="c", subcore_axis_name="s")


def _wids():
  cid = lax.axis_index("c")
  sid = lax.axis_index("s")
  return cid, sid, sid * NC + cid


def _gat_edge(src, dst, aS, aD, zden):
  EW = E_SD // NW
  nch = EW // C

  @functools.partial(
      pl.kernel, mesh=_sc_mesh(),
      compiler_params=pltpu.CompilerParams(needs_layout_passes=False, use_tc_tiling_on_sc=False),
      out_type=(jax.ShapeDtypeStruct((E_SD * HEADS,), f32),
                jax.ShapeDtypeStruct((NC, DEN_G), f32)),
      scratch_types=[
          pltpu.VMEM((N_SHIFT * HEADS,), f32),
          pltpu.VMEM((N_SHIFT * HEADS,), f32),
          pltpu.VMEM((C,), i32),
          pltpu.VMEM((C,), i32),
          pltpu.VMEM((C * HEADS,), f32),
          pltpu.VMEM((HEADS, C), f32),
          pltpu.VMEM((HEADS, C), i32),
          pltpu.VMEM_SHARED((DEN_G,), f32),
          pltpu.SemaphoreType.DMA,
      ])
  def body(src_hbm, dst_hbm, aS_hbm, aD_hbm, zden_hbm, ex_hbm, denP_hbm,
           aS_v, aD_v, src_c, dst_c, exc, exh, ibuf, den_sh, sem):
    cid, sid, wid = _wids()
    base = wid * EW
    pltpu.sync_copy(aS_hbm, aS_v)
    pltpu.sync_copy(aD_hbm, aD_v)

    @pl.when(sid == 0)
    def _():
      pltpu.sync_copy(zden_hbm, den_sh)
    plsc.subcore_barrier()

    def chunk(g, carry):
      eb = base + g * C
      pltpu.sync_copy(src_hbm.at[pl.ds(eb, C)], src_c)
      pltpu.sync_copy(dst_hbm.at[pl.ds(eb, C)], dst_c)
      for i in range(C // 16):
        loc16 = lax.iota(i32, 16) + i * 16
        s16 = src_c[pl.ds(i * 16, 16)]
        d16 = dst_c[pl.ds(i * 16, 16)]
        for h in range(HEADS):
          h16 = jnp.full((16,), h, i32)
          e16 = (plsc.load_gather(aS_v, [s16 * HEADS + h16])
                 + plsc.load_gather(aD_v, [d16 * HEADS + h16]))
          e16 = jnp.where(e16 >= 0.0, e16, 0.2 * e16)
          ex16 = jnp.exp(e16)
          plsc.store_scatter(exc, [loc16 * HEADS + h16], ex16)
          exh[h, pl.ds(i * 16, 16)] = ex16
          ibuf[h, pl.ds(i * 16, 16)] = d16 * HEADS + h16
      pltpu.sync_copy(exc, ex_hbm.at[pl.ds(eb * HEADS, C * HEADS)])
      for h in range(HEADS):
        pltpu.sync_copy(exh.at[h], den_sh.at[ibuf.at[h]], add=True)
      return carry

    lax.fori_loop(0, nch, chunk, 0)
    plsc.subcore_barrier()

    @pl.when(sid == 0)
    def _():
      pltpu.sync_copy(den_sh, denP_hbm.at[cid])

  return body(src, dst, aS, aD, zden)


def _gat_agg(src, dst, ex, denr, hg, zagg):
  EW = E_SD // NW
  nch = EW // C

  @functools.partial(
      pl.kernel, mesh=_sc_mesh(),
      compiler_params=pltpu.CompilerParams(needs_layout_passes=False, use_tc_tiling_on_sc=False),
      out_type=jax.ShapeDtypeStruct((NC, N_SHIFT, HID), f32),
      scratch_types=[
          pltpu.VMEM((DEN_G,), f32),
          pltpu.VMEM((C,), i32),
          pltpu.VMEM((C,), i32),
          pltpu.VMEM((C * HEADS,), f32),
          pltpu.VMEM((C, HEADS * HID), f32),
          pltpu.VMEM((C, HID), f32),
          pltpu.VMEM_SHARED((N_SHIFT, HID), f32),
          pltpu.SemaphoreType.DMA,
      ])
  def body(src_hbm, dst_hbm, ex_hbm, denr_hbm, hg_hbm, zagg_hbm, aggP_hbm,
           denr_v, src_c, dst_c, exc, rows, cvals, agg_sh, sem):
    cid, sid, wid = _wids()
    base = wid * EW
    pltpu.sync_copy(denr_hbm, denr_v)

    @pl.when(sid == 0)
    def _():
      pltpu.sync_copy(zagg_hbm, agg_sh)
    plsc.subcore_barrier()

    def chunk(g, carry):
      eb = base + g * C
      pltpu.sync_copy(src_hbm.at[pl.ds(eb, C)], src_c)
      pltpu.sync_copy(dst_hbm.at[pl.ds(eb, C)], dst_c)
      pltpu.sync_copy(ex_hbm.at[pl.ds(eb * HEADS, C * HEADS)], exc)
      pltpu.async_copy(hg_hbm.at[src_c], rows, sem).wait()
      for i in range(C // 16):
        e16 = lax.iota(i32, 16) + i * 16
        d16 = dst_c[pl.ds(i * 16, 16)]
        alphas = []
        for h in range(HEADS):
          h16 = jnp.full((16,), h, i32)
          exv = plsc.load_gather(exc, [e16 * HEADS + h16])
          drv = plsc.load_gather(denr_v, [d16 * HEADS + h16])
          alphas.append(exv * drv * 0.25)

        def colbody(ccol, carry2):
          c16 = jnp.zeros((16,), i32) + ccol
          acc = jnp.zeros((16,), f32)
          for h in range(HEADS):
            acc = acc + alphas[h] * plsc.load_gather(
                rows, [e16, c16 + h * HID])
          plsc.store_scatter(cvals, [e16, c16], acc)
          return carry2

        lax.fori_loop(0, HID, colbody, 0)
      pltpu.sync_copy(cvals, agg_sh.at[dst_c], add=True)
      return carry

    lax.fori_loop(0, nch, chunk, 0)
    plsc.subcore_barrier()

    @pl.when(sid == 0)
    def _():
      pltpu.sync_copy(agg_sh, aggP_hbm.at[cid])

  return body(src, dst, ex, denr, hg, zagg)


def _inj_score(dst, src, q, k, zden):
  EW = E_VI // NW
  nch = EW // C

  @functools.partial(
      pl.kernel, mesh=_sc_mesh(),
      compiler_params=pltpu.CompilerParams(needs_layout_passes=False, use_tc_tiling_on_sc=False),
      out_type=(jax.ShapeDtypeStruct((E_VI,), f32),
                jax.ShapeDtypeStruct((NC, DEN_I), f32)),
      scratch_types=[
          pltpu.VMEM((C,), i32),
          pltpu.VMEM((C,), i32),
          pltpu.VMEM((C, HID), f32),
          pltpu.VMEM((C, HID), f32),
          pltpu.VMEM((C,), f32),
          pltpu.VMEM_SHARED((DEN_I,), f32),
          pltpu.SemaphoreType.DMA,
      ])
  def body(dst_hbm, src_hbm, q_hbm, k_hbm, zden_hbm, ex_hbm, denP_hbm,
           dst_c, src_c, qrows, krows, sbuf, den_sh, sem):
    cid, sid, wid = _wids()
    base = wid * EW

    @pl.when(sid == 0)
    def _():
      pltpu.sync_copy(zden_hbm, den_sh)
    plsc.subcore_barrier()

    def chunk(g, carry):
      eb = base + g * C
      pltpu.sync_copy(dst_hbm.at[pl.ds(eb, C)], dst_c)
      pltpu.sync_copy(src_hbm.at[pl.ds(eb, C)], src_c)
      pltpu.async_copy(q_hbm.at[dst_c], qrows, sem).wait()
      pltpu.async_copy(k_hbm.at[src_c], krows, sem).wait()
      for i in range(C // 16):
        e16 = lax.iota(i32, 16) + i * 16

        def colbody(ccol, acc):
          c16 = jnp.zeros((16,), i32) + ccol
          return acc + (plsc.load_gather(qrows, [e16, c16])
                        * plsc.load_gather(krows, [e16, c16]))

        acc = lax.fori_loop(0, HID, colbody, jnp.zeros((16,), f32))
        sbuf[pl.ds(i * 16, 16)] = jnp.exp(acc * (1.0 / SCALE))
      pltpu.sync_copy(sbuf, ex_hbm.at[pl.ds(eb, C)])
      pltpu.sync_copy(sbuf, den_sh.at[dst_c], add=True)
      return carry

    lax.fori_loop(0, nch, chunk, 0)
    plsc.subcore_barrier()

    @pl.when(sid == 0)
    def _():
      pltpu.sync_copy(den_sh, denP_hbm.at[cid])

  return body(dst, src, q, k, zden)


def _alphaize(dst, ex, denr):
  EW = E_VI // NW
  nch = EW // C

  @functools.partial(
      pl.kernel, mesh=_sc_mesh(),
      compiler_params=pltpu.CompilerParams(needs_layout_passes=False, use_tc_tiling_on_sc=False),
      out_type=jax.ShapeDtypeStruct((E_VI,), f32),
      scratch_types=[
          pltpu.VMEM((DEN_I,), f32),
          pltpu.VMEM((C,), i32),
          pltpu.VMEM((C,), f32),
          pltpu.SemaphoreType.DMA,
      ])
  def body(dst_hbm, ex_hbm, denr_hbm, al_hbm, denr_v, dst_c, buf, sem):
    cid, sid, wid = _wids()
    base = wid * EW
    pltpu.sync_copy(denr_hbm, denr_v)

    def chunk(g, carry):
      eb = base + g * C
      pltpu.sync_copy(dst_hbm.at[pl.ds(eb, C)], dst_c)
      pltpu.sync_copy(ex_hbm.at[pl.ds(eb, C)], buf)
      for i in range(C // 16):
        sl = pl.ds(i * 16, 16)
        buf[sl] = buf[sl] * plsc.load_gather(denr_v, [dst_c[sl]])
      pltpu.sync_copy(buf, al_hbm.at[pl.ds(eb, C)])
      return carry

    lax.fori_loop(0, nch, chunk, 0)

  return body(dst, ex, denr)


def _inj_agg(dst, src, alpha, vals, zmsg):
  CI = 160
  SB = 80
  EW = E_VI // NS
  nch = EW // CI

  @functools.partial(
      pl.kernel, mesh=_sc_mesh(),
      compiler_params=pltpu.CompilerParams(needs_layout_passes=False, use_tc_tiling_on_sc=False),
      out_type=jax.ShapeDtypeStruct((N_VAR, HID), f32),
      scratch_types=[
          pltpu.VMEM((CI,), i32), pltpu.VMEM((CI,), i32),
          pltpu.VMEM((CI,), i32), pltpu.VMEM((CI,), i32),
          pltpu.VMEM((CI,), f32), pltpu.VMEM((CI,), f32),
          pltpu.VMEM((CI, HID), f32), pltpu.VMEM((CI, HID), f32),
          pltpu.VMEM((CI,), f32),
          pltpu.VMEM((CI // SB, SB), i32),
          pltpu.VMEM_SHARED((RNG, HID), f32),
          pltpu.SemaphoreType.DMA, pltpu.SemaphoreType.DMA,
      ])
  def body(dst_hbm, src_hbm, al_hbm, vals_hbm, zmsg_hbm, msg_hbm,
           dst0, dst1, src0, src1, al0, al1, rows0, rows1,
           abuf, ib2, msg_sh, sem0, sem1):
    cid, sid, wid = _wids()
    base = sid * EW
    sets = ((dst0, src0, al0, rows0, sem0),
            (dst1, src1, al1, rows1, sem1))

    def prefetch(g, st):
      dstX, srcX, alX, rowsX, semX = st
      eb = base + g * CI
      pltpu.sync_copy(dst_hbm.at[pl.ds(eb, CI)], dstX)
      pltpu.sync_copy(src_hbm.at[pl.ds(eb, CI)], srcX)
      pltpu.sync_copy(al_hbm.at[pl.ds(eb, CI)], alX)
      for b in range(CI // SB):
        pltpu.async_copy(vals_hbm.at[srcX.at[pl.ds(b * SB, SB)]],
                         rowsX.at[pl.ds(b * SB, SB)], semX)

    def drain(st):
      dstX, srcX, alX, rowsX, semX = st
      for b in range(CI // SB):
        pltpu.make_async_copy(vals_hbm.at[srcX.at[pl.ds(b * SB, SB)]],
                              rowsX.at[pl.ds(b * SB, SB)], semX).wait()

    def compute(rb, st):
      dstX, srcX, alX, rowsX, semX = st
      for i in range(CI // 16):
        sl = pl.ds(i * 16, 16)
        d16 = dstX[sl]
        inr = (d16 >= rb) & (d16 < rb + RNG)
        abuf[sl] = jnp.where(inr, alX[sl], 0.0)
        ib2[i // (SB // 16), pl.ds((i % (SB // 16)) * 16, 16)] = (
            jnp.where(inr, d16 - rb, 0))

      def colbody(ccol, carry2):
        c16 = jnp.zeros((16,), i32) + ccol
        for i in range(CI // 16):
          e16 = lax.iota(i32, 16) + i * 16
          v16 = plsc.load_gather(rowsX, [e16, c16])
          plsc.store_scatter(rowsX, [e16, c16],
                             v16 * abuf[pl.ds(i * 16, 16)])
        return carry2

      lax.fori_loop(0, HID, colbody, 0)
      for b in range(CI // SB):
        pltpu.sync_copy(rowsX.at[pl.ds(b * SB, SB)],
                        msg_sh.at[ib2.at[b]], add=True)

    for r in range(2):
      rb = (lax.axis_index("c") * 2 + r) * RNG

      @pl.when(sid == 0)
      def _():
        pltpu.sync_copy(zmsg_hbm, msg_sh)
      plsc.subcore_barrier()

      prefetch(0, sets[0])

      def pairbody(g2, carry):
        gA = 2 * g2
        prefetch(gA + 1, sets[1])
        drain(sets[0])
        compute(rb, sets[0])
        prefetch(jnp.minimum(gA + 2, nch - 1), sets[0])
        drain(sets[1])
        compute(rb, sets[1])
        return carry

      lax.fori_loop(0, nch // 2, pairbody, 0)
      drain(sets[0])
      plsc.subcore_barrier()

      @pl.when(sid == 0)
      def _():
        pltpu.sync_copy(msg_sh, msg_hbm.at[pl.ds(rb, RNG)])
      plsc.subcore_barrier()

  return body(dst, src, alpha, vals, zmsg)


def kernel(x_employee, x_shift, x_variable, x_constraint, edge_same_day,
           edge_var_emp, edge_var_shift, W_proj_emp, b_proj_emp,
           W_proj_shift, b_proj_shift, W_proj_var, b_proj_var, W_proj_con,
           b_proj_con, W_gat, att_src, att_dst, b_gat, W_inj_emp, b_inj_emp,
           W_inj_shift, b_inj_shift, Wq_emp, Wk_emp, Wq_shift, Wk_shift,
           W_fuse, b_fuse):
  del x_constraint, W_proj_con, b_proj_con

  A_src = jnp.zeros((HEADS * HID, HEADS), f32)
  A_dst = jnp.zeros((HEADS * HID, HEADS), f32)
  for h in range(HEADS):
    A_src = A_src.at[h * HID:(h + 1) * HID, h].set(att_src[h])
    A_dst = A_dst.at[h * HID:(h + 1) * HID, h].set(att_dst[h])

  h_emp, k_emp, v_emp = _ent_pre(x_employee, W_proj_emp, b_proj_emp,
                                 Wk_emp, W_inj_emp, b_inj_emp)
  del h_emp
  h_shift0, hg, aS, aD = _shift_pre(x_shift, W_proj_shift, b_proj_shift,
                                    W_gat, A_src, A_dst)
  h_var, q_emp, q_shift = _var_pre(x_variable, W_proj_var, b_proj_var,
                                   Wq_emp, Wq_shift)

  src_sd = edge_same_day[0]
  dst_sd = edge_same_day[1]
  ex_g, denP_g = _gat_edge(src_sd, dst_sd, aS.reshape(-1), aD.reshape(-1),
                           jnp.zeros((DEN_G,), f32))
  denr_g = _denr(denP_g[0], denP_g[1], 1e-16)
  aggP = _gat_agg(src_sd, dst_sd, ex_g, denr_g, hg,
                  jnp.zeros((N_SHIFT, HID), f32))
  k_shift, v_shift = _post_gat(aggP[0], aggP[1], b_gat, h_shift0,
                               Wk_shift, W_inj_shift, b_inj_shift)

  zden_i = jnp.zeros((DEN_I,), f32)
  zmsg = jnp.zeros((RNG, HID), f32)

  d_ve = edge_var_emp[0]
  s_ve = edge_var_emp[1]
  ex_e, denP_e = _inj_score(d_ve, s_ve, q_emp, k_emp, zden_i)
  denr_e = _denr(denP_e[0], denP_e[1], 1e-9)
  al_e = _alphaize(d_ve, ex_e, denr_e)
  msg_e = _inj_agg(d_ve, s_ve, al_e, v_emp, zmsg)

  d_vs = edge_var_shift[0]
  s_vs = edge_var_shift[1]
  ex_s, denP_s = _inj_score(d_vs, s_vs, q_shift, k_shift, zden_i)
  denr_s = _denr(denP_s[0], denP_s[1], 1e-9)
  al_s = _alphaize(d_vs, ex_s, denr_s)
  msg_s = _inj_agg(d_vs, s_vs, al_s, v_shift, zmsg)

  W1 = W_fuse[:HID]
  W2 = W_fuse[HID:2 * HID]
  W3 = W_fuse[2 * HID:]
  return _fuse(h_var, msg_e, msg_s, W1, W2, W3, b_fuse)


# speedup vs baseline: 7.8690x; 2.0637x over previous
import functools

import jax
import jax.numpy as jnp
from jax import lax
from jax.experimental import pallas as pl
from jax.experimental.pallas import tpu as pltpu
from jax.experimental.pallas import tpu_sc as plsc

HID = 64
HEADS = 4
SCALE = 8.0
NC = 2
NS = 16
NW = NC * NS
C = 80

N_VAR = 100000
N_SHIFT = 10000
N_EMP = 10000
E_SD = 320000
E_VI = 640000
DEN_G = 40960
DEN_I = 100352
RNG = 25000

f32 = jnp.float32
i32 = jnp.int32


def _ent_pre(x, W, b, Wk, Wv, bv):
  N = x.shape[0]
  BR = 1000
  def body(x_ref, W_ref, b_ref, Wk_ref, Wv_ref, bv_ref, h_ref, k_ref, v_ref):
    h = jnp.maximum(x_ref[...] @ W_ref[...] + b_ref[...], 0.0)
    h_ref[...] = h
    k_ref[...] = h @ Wk_ref[...]
    v_ref[...] = h @ Wv_ref[...] + bv_ref[...]
  K = x.shape[1]
  return pl.pallas_call(
      body,
      grid=(N // BR,),
      in_specs=[
          pl.BlockSpec((BR, K), lambda i: (i, 0)),
          pl.BlockSpec((K, HID), lambda i: (0, 0)),
          pl.BlockSpec((1, HID), lambda i: (0, 0)),
          pl.BlockSpec((HID, HID), lambda i: (0, 0)),
          pl.BlockSpec((HID, HID), lambda i: (0, 0)),
          pl.BlockSpec((1, HID), lambda i: (0, 0)),
      ],
      out_specs=[
          pl.BlockSpec((BR, HID), lambda i: (i, 0)),
          pl.BlockSpec((BR, HID), lambda i: (i, 0)),
          pl.BlockSpec((BR, HID), lambda i: (i, 0)),
      ],
      out_shape=[jax.ShapeDtypeStruct((N, HID), f32)] * 3,
  )(x, W, b.reshape(1, HID), Wk, Wv, bv.reshape(1, HID))


def _shift_pre(x, W, b, W_gat, A_src, A_dst):
  N = x.shape[0]
  BR = 1000
  K = x.shape[1]
  def body(x_ref, W_ref, b_ref, Wg_ref, As_ref, Ad_ref,
           h_ref, hg_ref, as_ref, ad_ref):
    h = jnp.maximum(x_ref[...] @ W_ref[...] + b_ref[...], 0.0)
    hg = h @ Wg_ref[...]
    h_ref[...] = h
    hg_ref[...] = hg
    as_ref[...] = hg @ As_ref[...]
    ad_ref[...] = hg @ Ad_ref[...]
  return pl.pallas_call(
      body,
      grid=(N // BR,),
      in_specs=[
          pl.BlockSpec((BR, K), lambda i: (i, 0)),
          pl.BlockSpec((K, HID), lambda i: (0, 0)),
          pl.BlockSpec((1, HID), lambda i: (0, 0)),
          pl.BlockSpec((HID, HEADS * HID), lambda i: (0, 0)),
          pl.BlockSpec((HEADS * HID, HEADS), lambda i: (0, 0)),
          pl.BlockSpec((HEADS * HID, HEADS), lambda i: (0, 0)),
      ],
      out_specs=[
          pl.BlockSpec((BR, HID), lambda i: (i, 0)),
          pl.BlockSpec((BR, HEADS * HID), lambda i: (i, 0)),
          pl.BlockSpec((BR, HEADS), lambda i: (i, 0)),
          pl.BlockSpec((BR, HEADS), lambda i: (i, 0)),
      ],
      out_shape=[
          jax.ShapeDtypeStruct((N, HID), f32),
          jax.ShapeDtypeStruct((N, HEADS * HID), f32),
          jax.ShapeDtypeStruct((N, HEADS), f32),
          jax.ShapeDtypeStruct((N, HEADS), f32),
      ],
  )(x, W, b.reshape(1, HID), W_gat, A_src, A_dst)


def _var_pre(x, W, b, Wq1, Wq2):
  N = x.shape[0]
  BR = 1000
  K = x.shape[1]
  def body(x_ref, W_ref, b_ref, W1_ref, W2_ref, h_ref, q1_ref, q2_ref):
    h = jnp.maximum(x_ref[...] @ W_ref[...] + b_ref[...], 0.0)
    h_ref[...] = h
    q1_ref[...] = h @ W1_ref[...]
    q2_ref[...] = h @ W2_ref[...]
  return pl.pallas_call(
      body,
      grid=(N // BR,),
      in_specs=[
          pl.BlockSpec((BR, K), lambda i: (i, 0)),
          pl.BlockSpec((K, HID), lambda i: (0, 0)),
          pl.BlockSpec((1, HID), lambda i: (0, 0)),
          pl.BlockSpec((HID, HID), lambda i: (0, 0)),
          pl.BlockSpec((HID, HID), lambda i: (0, 0)),
      ],
      out_specs=[
          pl.BlockSpec((BR, HID), lambda i: (i, 0)),
          pl.BlockSpec((BR, HID), lambda i: (i, 0)),
          pl.BlockSpec((BR, HID), lambda i: (i, 0)),
      ],
      out_shape=[jax.ShapeDtypeStruct((N, HID), f32)] * 3,
  )(x, W, b.reshape(1, HID), Wq1, Wq2)


def _denr(d0, d1, clip):
  M = d0.shape[0]
  rows = M // 128
  def body(a_ref, b_ref, o_ref):
    o_ref[...] = 1.0 / jnp.maximum(a_ref[...] + b_ref[...], clip)
  out = pl.pallas_call(
      body,
      out_shape=jax.ShapeDtypeStruct((rows, 128), f32),
  )(d0.reshape(rows, 128), d1.reshape(rows, 128))
  return out.reshape(M)


def _post_gat(agg0, agg1, b_gat, h0, Wk, Wv, bv):
  N = h0.shape[0]
  BR = 1000
  def body(a0_ref, a1_ref, bg_ref, h0_ref, Wk_ref, Wv_ref, bv_ref,
           k_ref, v_ref):
    g = a0_ref[...] + a1_ref[...] + bg_ref[...]
    hs = jnp.maximum(g, 0.0) + h0_ref[...]
    k_ref[...] = hs @ Wk_ref[...]
    v_ref[...] = hs @ Wv_ref[...] + bv_ref[...]
  return pl.pallas_call(
      body,
      grid=(N // BR,),
      in_specs=[
          pl.BlockSpec((BR, HID), lambda i: (i, 0)),
          pl.BlockSpec((BR, HID), lambda i: (i, 0)),
          pl.BlockSpec((1, HID), lambda i: (0, 0)),
          pl.BlockSpec((BR, HID), lambda i: (i, 0)),
          pl.BlockSpec((HID, HID), lambda i: (0, 0)),
          pl.BlockSpec((HID, HID), lambda i: (0, 0)),
          pl.BlockSpec((1, HID), lambda i: (0, 0)),
      ],
      out_specs=[
          pl.BlockSpec((BR, HID), lambda i: (i, 0)),
          pl.BlockSpec((BR, HID), lambda i: (i, 0)),
      ],
      out_shape=[jax.ShapeDtypeStruct((N, HID), f32)] * 2,
  )(agg0, agg1, b_gat.reshape(1, HID), h0, Wk, Wv, bv.reshape(1, HID))


def _fuse(hv, mE, mS, W1, W2, W3, b):
  N = hv.shape[0]
  BR = 1000
  def body(hv_ref, mE_ref, mS_ref, W1_ref, W2_ref, W3_ref, b_ref, o_ref):
    z = (hv_ref[...] @ W1_ref[...] + mE_ref[...] @ W2_ref[...]
         + mS_ref[...] @ W3_ref[...] + b_ref[...])
    z = jnp.maximum(z, 0.0)
    o_ref[...] = jnp.sqrt(jnp.sum(z * z, axis=1, keepdims=True))
  out = pl.pallas_call(
      body,
      grid=(N // BR,),
      in_specs=[
          pl.BlockSpec((BR, HID), lambda i: (i, 0)),
          pl.BlockSpec((BR, HID), lambda i: (i, 0)),
          pl.BlockSpec((BR, HID), lambda i: (i, 0)),
          pl.BlockSpec((HID, HID), lambda i: (0, 0)),
          pl.BlockSpec((HID, HID), lambda i: (0, 0)),
          pl.BlockSpec((HID, HID), lambda i: (0, 0)),
          pl.BlockSpec((1, HID), lambda i: (0, 0)),
      ],
      out_specs=pl.BlockSpec((BR, 1), lambda i: (i, 0)),
      out_shape=jax.ShapeDtypeStruct((N, 1), f32),
  )(hv, mE, mS, W1, W2, W3, b.reshape(1, HID))
  return out.reshape(N)


def _sc_mesh():
  return plsc.VectorSubcoreMesh(core_axis_name="c", subcore_axis_name="s")


def _wids():
  cid = lax.axis_index("c")
  sid = lax.axis_index("s")
  return cid, sid, sid * NC + cid


def _gat_edge(src, dst, aS, aD, zden):
  EW = E_SD // NW
  nch = EW // C

  @functools.partial(
      pl.kernel, mesh=_sc_mesh(),
      compiler_params=pltpu.CompilerParams(needs_layout_passes=False, use_tc_tiling_on_sc=False),
      out_type=(jax.ShapeDtypeStruct((E_SD * HEADS,), f32),
                jax.ShapeDtypeStruct((NC, DEN_G), f32)),
      scratch_types=[
          pltpu.VMEM((N_SHIFT * HEADS,), f32),
          pltpu.VMEM((N_SHIFT * HEADS,), f32),
          pltpu.VMEM((C,), i32),
          pltpu.VMEM((C,), i32),
          pltpu.VMEM((C * HEADS,), f32),
          pltpu.VMEM((HEADS, C), f32),
          pltpu.VMEM((HEADS, C), i32),
          pltpu.VMEM_SHARED((DEN_G,), f32),
          pltpu.SemaphoreType.DMA,
      ])
  def body(src_hbm, dst_hbm, aS_hbm, aD_hbm, zden_hbm, ex_hbm, denP_hbm,
           aS_v, aD_v, src_c, dst_c, exc, exh, ibuf, den_sh, sem):
    cid, sid, wid = _wids()
    base = wid * EW
    pltpu.sync_copy(aS_hbm, aS_v)
    pltpu.sync_copy(aD_hbm, aD_v)

    @pl.when(sid == 0)
    def _():
      pltpu.sync_copy(zden_hbm, den_sh)
    plsc.subcore_barrier()

    def chunk(g, carry):
      eb = base + g * C
      pltpu.sync_copy(src_hbm.at[pl.ds(eb, C)], src_c)
      pltpu.sync_copy(dst_hbm.at[pl.ds(eb, C)], dst_c)
      for i in range(C // 16):
        loc16 = lax.iota(i32, 16) + i * 16
        s16 = src_c[pl.ds(i * 16, 16)]
        d16 = dst_c[pl.ds(i * 16, 16)]
        for h in range(HEADS):
          h16 = jnp.full((16,), h, i32)
          e16 = (plsc.load_gather(aS_v, [s16 * HEADS + h16])
                 + plsc.load_gather(aD_v, [d16 * HEADS + h16]))
          e16 = jnp.where(e16 >= 0.0, e16, 0.2 * e16)
          ex16 = jnp.exp(e16)
          plsc.store_scatter(exc, [loc16 * HEADS + h16], ex16)
          exh[h, pl.ds(i * 16, 16)] = ex16
          ibuf[h, pl.ds(i * 16, 16)] = d16 * HEADS + h16
      pltpu.sync_copy(exc, ex_hbm.at[pl.ds(eb * HEADS, C * HEADS)])
      for h in range(HEADS):
        pltpu.sync_copy(exh.at[h], den_sh.at[ibuf.at[h]], add=True)
      return carry

    lax.fori_loop(0, nch, chunk, 0)
    plsc.subcore_barrier()

    @pl.when(sid == 0)
    def _():
      pltpu.sync_copy(den_sh, denP_hbm.at[cid])

  return body(src, dst, aS, aD, zden)


def _gat_agg(src, dst, ex, denr, hg, zagg):
  EW = E_SD // NW
  nch = EW // C

  @functools.partial(
      pl.kernel, mesh=_sc_mesh(),
      compiler_params=pltpu.CompilerParams(needs_layout_passes=False, use_tc_tiling_on_sc=False),
      out_type=jax.ShapeDtypeStruct((NC, N_SHIFT, HID), f32),
      scratch_types=[
          pltpu.VMEM((DEN_G,), f32),
          pltpu.VMEM((C,), i32),
          pltpu.VMEM((C,), i32),
          pltpu.VMEM((C * HEADS,), f32),
          pltpu.VMEM((C, HEADS * HID), f32),
          pltpu.VMEM((C, HID), f32),
          pltpu.VMEM_SHARED((N_SHIFT, HID), f32),
          pltpu.SemaphoreType.DMA,
      ])
  def body(src_hbm, dst_hbm, ex_hbm, denr_hbm, hg_hbm, zagg_hbm, aggP_hbm,
           denr_v, src_c, dst_c, exc, rows, cvals, agg_sh, sem):
    cid, sid, wid = _wids()
    base = wid * EW
    pltpu.sync_copy(denr_hbm, denr_v)

    @pl.when(sid == 0)
    def _():
      pltpu.sync_copy(zagg_hbm, agg_sh)
    plsc.subcore_barrier()

    def chunk(g, carry):
      eb = base + g * C
      pltpu.sync_copy(src_hbm.at[pl.ds(eb, C)], src_c)
      pltpu.sync_copy(dst_hbm.at[pl.ds(eb, C)], dst_c)
      pltpu.sync_copy(ex_hbm.at[pl.ds(eb * HEADS, C * HEADS)], exc)
      pltpu.async_copy(hg_hbm.at[src_c], rows, sem).wait()
      for i in range(C // 16):
        e16 = lax.iota(i32, 16) + i * 16
        d16 = dst_c[pl.ds(i * 16, 16)]
        alphas = []
        for h in range(HEADS):
          h16 = jnp.full((16,), h, i32)
          exv = plsc.load_gather(exc, [e16 * HEADS + h16])
          drv = plsc.load_gather(denr_v, [d16 * HEADS + h16])
          alphas.append(exv * drv * 0.25)

        def colbody(ccol, carry2):
          c16 = jnp.zeros((16,), i32) + ccol
          acc = jnp.zeros((16,), f32)
          for h in range(HEADS):
            acc = acc + alphas[h] * plsc.load_gather(
                rows, [e16, c16 + h * HID])
          plsc.store_scatter(cvals, [e16, c16], acc)
          return carry2

        lax.fori_loop(0, HID, colbody, 0)
      pltpu.sync_copy(cvals, agg_sh.at[dst_c], add=True)
      return carry

    lax.fori_loop(0, nch, chunk, 0)
    plsc.subcore_barrier()

    @pl.when(sid == 0)
    def _():
      pltpu.sync_copy(agg_sh, aggP_hbm.at[cid])

  return body(src, dst, ex, denr, hg, zagg)


def _inj_score(dst, src, q, k, zden):
  EW = E_VI // NW
  nch = EW // C

  @functools.partial(
      pl.kernel, mesh=_sc_mesh(),
      compiler_params=pltpu.CompilerParams(needs_layout_passes=False, use_tc_tiling_on_sc=False),
      out_type=(jax.ShapeDtypeStruct((E_VI,), f32),
                jax.ShapeDtypeStruct((NC, DEN_I), f32)),
      scratch_types=[
          pltpu.VMEM((C,), i32),
          pltpu.VMEM((C,), i32),
          pltpu.VMEM((C, HID), f32),
          pltpu.VMEM((C, HID), f32),
          pltpu.VMEM((C,), f32),
          pltpu.VMEM_SHARED((DEN_I,), f32),
          pltpu.SemaphoreType.DMA,
      ])
  def body(dst_hbm, src_hbm, q_hbm, k_hbm, zden_hbm, ex_hbm, denP_hbm,
           dst_c, src_c, qrows, krows, sbuf, den_sh, sem):
    cid, sid, wid = _wids()
    base = wid * EW

    @pl.when(sid == 0)
    def _():
      pltpu.sync_copy(zden_hbm, den_sh)
    plsc.subcore_barrier()

    def chunk(g, carry):
      eb = base + g * C
      pltpu.sync_copy(dst_hbm.at[pl.ds(eb, C)], dst_c)
      pltpu.sync_copy(src_hbm.at[pl.ds(eb, C)], src_c)
      pltpu.async_copy(q_hbm.at[dst_c], qrows, sem).wait()
      pltpu.async_copy(k_hbm.at[src_c], krows, sem).wait()
      for i in range(C // 16):
        e16 = lax.iota(i32, 16) + i * 16

        def colbody(ccol, acc):
          c16 = jnp.zeros((16,), i32) + ccol
          return acc + (plsc.load_gather(qrows, [e16, c16])
                        * plsc.load_gather(krows, [e16, c16]))

        acc = lax.fori_loop(0, HID, colbody, jnp.zeros((16,), f32))
        sbuf[pl.ds(i * 16, 16)] = jnp.exp(acc * (1.0 / SCALE))
      pltpu.sync_copy(sbuf, ex_hbm.at[pl.ds(eb, C)])
      pltpu.sync_copy(sbuf, den_sh.at[dst_c], add=True)
      return carry

    lax.fori_loop(0, nch, chunk, 0)
    plsc.subcore_barrier()

    @pl.when(sid == 0)
    def _():
      pltpu.sync_copy(den_sh, denP_hbm.at[cid])

  return body(dst, src, q, k, zden)


def _alphaize(dst, ex, denr):
  EW = E_VI // NW
  nch = EW // C

  @functools.partial(
      pl.kernel, mesh=_sc_mesh(),
      compiler_params=pltpu.CompilerParams(needs_layout_passes=False, use_tc_tiling_on_sc=False),
      out_type=jax.ShapeDtypeStruct((E_VI,), f32),
      scratch_types=[
          pltpu.VMEM((DEN_I,), f32),
          pltpu.VMEM((C,), i32),
          pltpu.VMEM((C,), f32),
          pltpu.SemaphoreType.DMA,
      ])
  def body(dst_hbm, ex_hbm, denr_hbm, al_hbm, denr_v, dst_c, buf, sem):
    cid, sid, wid = _wids()
    base = wid * EW
    pltpu.sync_copy(denr_hbm, denr_v)

    def chunk(g, carry):
      eb = base + g * C
      pltpu.sync_copy(dst_hbm.at[pl.ds(eb, C)], dst_c)
      pltpu.sync_copy(ex_hbm.at[pl.ds(eb, C)], buf)
      for i in range(C // 16):
        sl = pl.ds(i * 16, 16)
        buf[sl] = buf[sl] * plsc.load_gather(denr_v, [dst_c[sl]])
      pltpu.sync_copy(buf, al_hbm.at[pl.ds(eb, C)])
      return carry

    lax.fori_loop(0, nch, chunk, 0)

  return body(dst, ex, denr)


def _inj_agg(dst, src, alpha, vals_cs, zmsg):
  CI = 160
  SB = 80
  NG = 16
  N_ENT = 10000
  EW = E_VI // NS
  nch = EW // CI

  @functools.partial(
      pl.kernel, mesh=_sc_mesh(),
      compiler_params=pltpu.CompilerParams(needs_layout_passes=False, use_tc_tiling_on_sc=False),
      out_type=jax.ShapeDtypeStruct((4 * N_VAR, NG), f32),
      scratch_types=[
          pltpu.VMEM((CI,), i32), pltpu.VMEM((CI,), i32),
          pltpu.VMEM((CI,), i32), pltpu.VMEM((CI,), i32),
          pltpu.VMEM((CI,), f32), pltpu.VMEM((CI,), f32),
          pltpu.VMEM((CI, NG), f32), pltpu.VMEM((CI, NG), f32),
          pltpu.VMEM((CI,), i32),
          pltpu.VMEM((CI // SB, SB), i32),
          pltpu.VMEM_SHARED((N_VAR, NG), f32),
          pltpu.SemaphoreType.DMA, pltpu.SemaphoreType.DMA,
      ])
  def body(dst_hbm, src_hbm, al_hbm, vals_hbm, zmsg_hbm, msg_hbm,
           dst0, dst1, sidx0, sidx1, al0, al1, rows0, rows1,
           srcb, ib2, msg_sh, sem0, sem1):
    cid, sid, wid = _wids()
    base = sid * EW
    sets = ((dst0, sidx0, al0, rows0, sem0),
            (dst1, sidx1, al1, rows1, sem1))

    def prefetch(g, st, sbase):
      dstX, sidxX, alX, rowsX, semX = st
      eb = base + g * CI
      pltpu.sync_copy(dst_hbm.at[pl.ds(eb, CI)], dstX)
      pltpu.sync_copy(src_hbm.at[pl.ds(eb, CI)], srcb)
      pltpu.sync_copy(al_hbm.at[pl.ds(eb, CI)], alX)
      for i in range(CI // 16):
        sl = pl.ds(i * 16, 16)
        sidxX[sl] = srcb[sl] + sbase
      for b in range(CI // SB):
        pltpu.async_copy(vals_hbm.at[sidxX.at[pl.ds(b * SB, SB)]],
                         rowsX.at[pl.ds(b * SB, SB)], semX)

    def drain(st):
      dstX, sidxX, alX, rowsX, semX = st
      for b in range(CI // SB):
        pltpu.make_async_copy(vals_hbm.at[sidxX.at[pl.ds(b * SB, SB)]],
                              rowsX.at[pl.ds(b * SB, SB)], semX).wait()

    def compute(st):
      dstX, sidxX, alX, rowsX, semX = st
      for i in range(CI // 16):
        sl = pl.ds(i * 16, 16)
        ib2[i // (SB // 16), pl.ds((i % (SB // 16)) * 16, 16)] = dstX[sl]

      def colbody(ccol, carry2):
        c16 = jnp.zeros((16,), i32) + ccol
        for i in range(CI // 16):
          e16 = lax.iota(i32, 16) + i * 16
          v16 = plsc.load_gather(rowsX, [e16, c16])
          plsc.store_scatter(rowsX, [e16, c16],
                             v16 * alX[pl.ds(i * 16, 16)])
        return carry2

      lax.fori_loop(0, NG, colbody, 0)
      for b in range(CI // SB):
        pltpu.sync_copy(rowsX.at[pl.ds(b * SB, SB)],
                        msg_sh.at[ib2.at[b]], add=True)

    for r in range(2):
      cg = lax.axis_index("c") * 2 + r
      sbase = cg * N_ENT

      @pl.when(sid == 0)
      def _():
        pltpu.sync_copy(zmsg_hbm, msg_sh)
      plsc.subcore_barrier()

      prefetch(0, sets[0], sbase)

      def pairbody(g2, carry):
        gA = 2 * g2
        prefetch(gA + 1, sets[1], sbase)
        drain(sets[0])
        compute(sets[0])
        prefetch(jnp.minimum(gA + 2, nch - 1), sets[0], sbase)
        drain(sets[1])
        compute(sets[1])
        return carry

      lax.fori_loop(0, nch // 2, pairbody, 0)
      drain(sets[0])
      plsc.subcore_barrier()

      @pl.when(sid == 0)
      def _():
        pltpu.sync_copy(msg_sh, msg_hbm.at[pl.ds(cg * N_VAR, N_VAR)])
      plsc.subcore_barrier()

  return body(dst, src, alpha, vals_cs, zmsg)


def kernel(x_employee, x_shift, x_variable, x_constraint, edge_same_day,
           edge_var_emp, edge_var_shift, W_proj_emp, b_proj_emp,
           W_proj_shift, b_proj_shift, W_proj_var, b_proj_var, W_proj_con,
           b_proj_con, W_gat, att_src, att_dst, b_gat, W_inj_emp, b_inj_emp,
           W_inj_shift, b_inj_shift, Wq_emp, Wk_emp, Wq_shift, Wk_shift,
           W_fuse, b_fuse):
  del x_constraint, W_proj_con, b_proj_con

  A_src = jnp.zeros((HEADS * HID, HEADS), f32)
  A_dst = jnp.zeros((HEADS * HID, HEADS), f32)
  for h in range(HEADS):
    A_src = A_src.at[h * HID:(h + 1) * HID, h].set(att_src[h])
    A_dst = A_dst.at[h * HID:(h + 1) * HID, h].set(att_dst[h])

  h_emp, k_emp, v_emp = _ent_pre(x_employee, W_proj_emp, b_proj_emp,
                                 Wk_emp, W_inj_emp, b_inj_emp)
  del h_emp
  h_shift0, hg, aS, aD = _shift_pre(x_shift, W_proj_shift, b_proj_shift,
                                    W_gat, A_src, A_dst)
  h_var, q_emp, q_shift = _var_pre(x_variable, W_proj_var, b_proj_var,
                                   Wq_emp, Wq_shift)

  src_sd = edge_same_day[0]
  dst_sd = edge_same_day[1]
  ex_g, denP_g = _gat_edge(src_sd, dst_sd, aS.reshape(-1), aD.reshape(-1),
                           jnp.zeros((DEN_G,), f32))
  denr_g = _denr(denP_g[0], denP_g[1], 1e-16)
  aggP = _gat_agg(src_sd, dst_sd, ex_g, denr_g, hg,
                  jnp.zeros((N_SHIFT, HID), f32))
  k_shift, v_shift = _post_gat(aggP[0], aggP[1], b_gat, h_shift0,
                               Wk_shift, W_inj_shift, b_inj_shift)

  zden_i = jnp.zeros((DEN_I,), f32)
  zmsg = jnp.zeros((N_VAR, 16), f32)

  def col_stack(v):
    return v.reshape(-1, 4, 16).transpose(1, 0, 2).reshape(-1, 16)

  def col_unstack(m):
    return m.reshape(4, N_VAR, 16).transpose(1, 0, 2).reshape(N_VAR, HID)

  d_ve = edge_var_emp[0]
  s_ve = edge_var_emp[1]
  ex_e, denP_e = _inj_score(d_ve, s_ve, q_emp, k_emp, zden_i)
  denr_e = _denr(denP_e[0], denP_e[1], 1e-9)
  al_e = _alphaize(d_ve, ex_e, denr_e)
  msg_e = col_unstack(_inj_agg(d_ve, s_ve, al_e, col_stack(v_emp), zmsg))

  d_vs = edge_var_shift[0]
  s_vs = edge_var_shift[1]
  ex_s, denP_s = _inj_score(d_vs, s_vs, q_shift, k_shift, zden_i)
  denr_s = _denr(denP_s[0], denP_s[1], 1e-9)
  al_s = _alphaize(d_vs, ex_s, denr_s)
  msg_s = col_unstack(_inj_agg(d_vs, s_vs, al_s, col_stack(v_shift), zmsg))

  W1 = W_fuse[:HID]
  W2 = W_fuse[HID:2 * HID]
  W3 = W_fuse[2 * HID:]
  return _fuse(h_var, msg_e, msg_s, W1, W2, W3, b_fuse)


# speedup vs baseline: 8.6811x; 1.1032x over previous
import functools

import jax
import jax.numpy as jnp
from jax import lax
from jax.experimental import pallas as pl
from jax.experimental.pallas import tpu as pltpu
from jax.experimental.pallas import tpu_sc as plsc

HID = 64
HEADS = 4
SCALE = 8.0
NC = 2
NS = 16
NW = NC * NS
C = 80

N_VAR = 100000
N_SHIFT = 10000
N_EMP = 10000
E_SD = 320000
E_VI = 640000
DEN_G = 40960
DEN_I = 100352
RNG = 25000

f32 = jnp.float32
i32 = jnp.int32


def _ent_pre(x, W, b, Wk, Wv, bv):
  N = x.shape[0]
  BR = 1000
  def body(x_ref, W_ref, b_ref, Wk_ref, Wv_ref, bv_ref, h_ref, k_ref, v_ref):
    h = jnp.maximum(x_ref[...] @ W_ref[...] + b_ref[...], 0.0)
    h_ref[...] = h
    k_ref[...] = h @ Wk_ref[...]
    v_ref[...] = h @ Wv_ref[...] + bv_ref[...]
  K = x.shape[1]
  return pl.pallas_call(
      body,
      grid=(N // BR,),
      in_specs=[
          pl.BlockSpec((BR, K), lambda i: (i, 0)),
          pl.BlockSpec((K, HID), lambda i: (0, 0)),
          pl.BlockSpec((1, HID), lambda i: (0, 0)),
          pl.BlockSpec((HID, HID), lambda i: (0, 0)),
          pl.BlockSpec((HID, HID), lambda i: (0, 0)),
          pl.BlockSpec((1, HID), lambda i: (0, 0)),
      ],
      out_specs=[
          pl.BlockSpec((BR, HID), lambda i: (i, 0)),
          pl.BlockSpec((BR, HID), lambda i: (i, 0)),
          pl.BlockSpec((BR, HID), lambda i: (i, 0)),
      ],
      out_shape=[jax.ShapeDtypeStruct((N, HID), f32)] * 3,
  )(x, W, b.reshape(1, HID), Wk, Wv, bv.reshape(1, HID))


def _shift_pre(x, W, b, W_gat, A_src, A_dst):
  N = x.shape[0]
  BR = 1000
  K = x.shape[1]
  def body(x_ref, W_ref, b_ref, Wg_ref, As_ref, Ad_ref,
           h_ref, hg_ref, as_ref, ad_ref):
    h = jnp.maximum(x_ref[...] @ W_ref[...] + b_ref[...], 0.0)
    hg = h @ Wg_ref[...]
    h_ref[...] = h
    hg_ref[...] = hg
    as_ref[...] = hg @ As_ref[...]
    ad_ref[...] = hg @ Ad_ref[...]
  return pl.pallas_call(
      body,
      grid=(N // BR,),
      in_specs=[
          pl.BlockSpec((BR, K), lambda i: (i, 0)),
          pl.BlockSpec((K, HID), lambda i: (0, 0)),
          pl.BlockSpec((1, HID), lambda i: (0, 0)),
          pl.BlockSpec((HID, HEADS * HID), lambda i: (0, 0)),
          pl.BlockSpec((HEADS * HID, HEADS), lambda i: (0, 0)),
          pl.BlockSpec((HEADS * HID, HEADS), lambda i: (0, 0)),
      ],
      out_specs=[
          pl.BlockSpec((BR, HID), lambda i: (i, 0)),
          pl.BlockSpec((BR, HEADS * HID), lambda i: (i, 0)),
          pl.BlockSpec((BR, HEADS), lambda i: (i, 0)),
          pl.BlockSpec((BR, HEADS), lambda i: (i, 0)),
      ],
      out_shape=[
          jax.ShapeDtypeStruct((N, HID), f32),
          jax.ShapeDtypeStruct((N, HEADS * HID), f32),
          jax.ShapeDtypeStruct((N, HEADS), f32),
          jax.ShapeDtypeStruct((N, HEADS), f32),
      ],
  )(x, W, b.reshape(1, HID), W_gat, A_src, A_dst)


def _var_pre(x, W, b, Wq1, Wq2):
  N = x.shape[0]
  BR = 1000
  K = x.shape[1]
  def body(x_ref, W_ref, b_ref, W1_ref, W2_ref, h_ref, q1_ref, q2_ref):
    h = jnp.maximum(x_ref[...] @ W_ref[...] + b_ref[...], 0.0)
    h_ref[...] = h
    q1_ref[...] = h @ W1_ref[...]
    q2_ref[...] = h @ W2_ref[...]
  return pl.pallas_call(
      body,
      grid=(N // BR,),
      in_specs=[
          pl.BlockSpec((BR, K), lambda i: (i, 0)),
          pl.BlockSpec((K, HID), lambda i: (0, 0)),
          pl.BlockSpec((1, HID), lambda i: (0, 0)),
          pl.BlockSpec((HID, HID), lambda i: (0, 0)),
          pl.BlockSpec((HID, HID), lambda i: (0, 0)),
      ],
      out_specs=[
          pl.BlockSpec((BR, HID), lambda i: (i, 0)),
          pl.BlockSpec((BR, HID), lambda i: (i, 0)),
          pl.BlockSpec((BR, HID), lambda i: (i, 0)),
      ],
      out_shape=[jax.ShapeDtypeStruct((N, HID), f32)] * 3,
  )(x, W, b.reshape(1, HID), Wq1, Wq2)


def _denr(d0, d1, clip):
  M = d0.shape[0]
  rows = M // 128
  def body(a_ref, b_ref, o_ref):
    o_ref[...] = 1.0 / jnp.maximum(a_ref[...] + b_ref[...], clip)
  out = pl.pallas_call(
      body,
      out_shape=jax.ShapeDtypeStruct((rows, 128), f32),
  )(d0.reshape(rows, 128), d1.reshape(rows, 128))
  return out.reshape(M)


def _post_gat(agg0, agg1, b_gat, h0, Wk, Wv, bv):
  N = h0.shape[0]
  BR = 1000
  def body(a0_ref, a1_ref, bg_ref, h0_ref, Wk_ref, Wv_ref, bv_ref,
           k_ref, v_ref):
    g = a0_ref[...] + a1_ref[...] + bg_ref[...]
    hs = jnp.maximum(g, 0.0) + h0_ref[...]
    k_ref[...] = hs @ Wk_ref[...]
    v_ref[...] = hs @ Wv_ref[...] + bv_ref[...]
  return pl.pallas_call(
      body,
      grid=(N // BR,),
      in_specs=[
          pl.BlockSpec((BR, HID), lambda i: (i, 0)),
          pl.BlockSpec((BR, HID), lambda i: (i, 0)),
          pl.BlockSpec((1, HID), lambda i: (0, 0)),
          pl.BlockSpec((BR, HID), lambda i: (i, 0)),
          pl.BlockSpec((HID, HID), lambda i: (0, 0)),
          pl.BlockSpec((HID, HID), lambda i: (0, 0)),
          pl.BlockSpec((1, HID), lambda i: (0, 0)),
      ],
      out_specs=[
          pl.BlockSpec((BR, HID), lambda i: (i, 0)),
          pl.BlockSpec((BR, HID), lambda i: (i, 0)),
      ],
      out_shape=[jax.ShapeDtypeStruct((N, HID), f32)] * 2,
  )(agg0, agg1, b_gat.reshape(1, HID), h0, Wk, Wv, bv.reshape(1, HID))


def _fuse(hv, mE, mS, W1, W2, W3, b):
  N = hv.shape[0]
  BR = 1000
  def body(hv_ref, mE_ref, mS_ref, W1_ref, W2_ref, W3_ref, b_ref, o_ref):
    z = (hv_ref[...] @ W1_ref[...] + mE_ref[...] @ W2_ref[...]
         + mS_ref[...] @ W3_ref[...] + b_ref[...])
    z = jnp.maximum(z, 0.0)
    o_ref[...] = jnp.sqrt(jnp.sum(z * z, axis=1, keepdims=True))
  out = pl.pallas_call(
      body,
      grid=(N // BR,),
      in_specs=[
          pl.BlockSpec((BR, HID), lambda i: (i, 0)),
          pl.BlockSpec((BR, HID), lambda i: (i, 0)),
          pl.BlockSpec((BR, HID), lambda i: (i, 0)),
          pl.BlockSpec((HID, HID), lambda i: (0, 0)),
          pl.BlockSpec((HID, HID), lambda i: (0, 0)),
          pl.BlockSpec((HID, HID), lambda i: (0, 0)),
          pl.BlockSpec((1, HID), lambda i: (0, 0)),
      ],
      out_specs=pl.BlockSpec((BR, 1), lambda i: (i, 0)),
      out_shape=jax.ShapeDtypeStruct((N, 1), f32),
  )(hv, mE, mS, W1, W2, W3, b.reshape(1, HID))
  return out.reshape(N)


def _sc_mesh():
  return plsc.VectorSubcoreMesh(core_axis_name="c", subcore_axis_name="s")


def _wids():
  cid = lax.axis_index("c")
  sid = lax.axis_index("s")
  return cid, sid, sid * NC + cid


def _gat_edge(src, dst, aS, aD, zden):
  EW = E_SD // NW
  nch = EW // C

  @functools.partial(
      pl.kernel, mesh=_sc_mesh(),
      compiler_params=pltpu.CompilerParams(needs_layout_passes=False, use_tc_tiling_on_sc=False),
      out_type=(jax.ShapeDtypeStruct((E_SD * HEADS,), f32),
                jax.ShapeDtypeStruct((NC, DEN_G), f32)),
      scratch_types=[
          pltpu.VMEM((N_SHIFT * HEADS,), f32),
          pltpu.VMEM((N_SHIFT * HEADS,), f32),
          pltpu.VMEM((C,), i32),
          pltpu.VMEM((C,), i32),
          pltpu.VMEM((C * HEADS,), f32),
          pltpu.VMEM((HEADS, C), f32),
          pltpu.VMEM((HEADS, C), i32),
          pltpu.VMEM_SHARED((DEN_G,), f32),
          pltpu.SemaphoreType.DMA,
      ])
  def body(src_hbm, dst_hbm, aS_hbm, aD_hbm, zden_hbm, ex_hbm, denP_hbm,
           aS_v, aD_v, src_c, dst_c, exc, exh, ibuf, den_sh, sem):
    cid, sid, wid = _wids()
    base = wid * EW
    pltpu.sync_copy(aS_hbm, aS_v)
    pltpu.sync_copy(aD_hbm, aD_v)

    @pl.when(sid == 0)
    def _():
      pltpu.sync_copy(zden_hbm, den_sh)
    plsc.subcore_barrier()

    def chunk(g, carry):
      eb = base + g * C
      pltpu.sync_copy(src_hbm.at[pl.ds(eb, C)], src_c)
      pltpu.sync_copy(dst_hbm.at[pl.ds(eb, C)], dst_c)
      for i in range(C // 16):
        loc16 = lax.iota(i32, 16) + i * 16
        s16 = src_c[pl.ds(i * 16, 16)]
        d16 = dst_c[pl.ds(i * 16, 16)]
        for h in range(HEADS):
          h16 = jnp.full((16,), h, i32)
          e16 = (plsc.load_gather(aS_v, [s16 * HEADS + h16])
                 + plsc.load_gather(aD_v, [d16 * HEADS + h16]))
          e16 = jnp.where(e16 >= 0.0, e16, 0.2 * e16)
          ex16 = jnp.exp(e16)
          plsc.store_scatter(exc, [loc16 * HEADS + h16], ex16)
          exh[h, pl.ds(i * 16, 16)] = ex16
          ibuf[h, pl.ds(i * 16, 16)] = d16 * HEADS + h16
      pltpu.sync_copy(exc, ex_hbm.at[pl.ds(eb * HEADS, C * HEADS)])
      for h in range(HEADS):
        pltpu.sync_copy(exh.at[h], den_sh.at[ibuf.at[h]], add=True)
      return carry

    lax.fori_loop(0, nch, chunk, 0)
    plsc.subcore_barrier()

    @pl.when(sid == 0)
    def _():
      pltpu.sync_copy(den_sh, denP_hbm.at[cid])

  return body(src, dst, aS, aD, zden)


def _gat_agg(src, dst, ex, denr, hg, zagg):
  EW = E_SD // NW
  nch = EW // C

  @functools.partial(
      pl.kernel, mesh=_sc_mesh(),
      compiler_params=pltpu.CompilerParams(needs_layout_passes=False, use_tc_tiling_on_sc=False),
      out_type=jax.ShapeDtypeStruct((NC, N_SHIFT, HID), f32),
      scratch_types=[
          pltpu.VMEM((DEN_G,), f32),
          pltpu.VMEM((C,), i32),
          pltpu.VMEM((C,), i32),
          pltpu.VMEM((C * HEADS,), f32),
          pltpu.VMEM((C, HEADS * HID), f32),
          pltpu.VMEM((C, HID), f32),
          pltpu.VMEM_SHARED((N_SHIFT, HID), f32),
          pltpu.SemaphoreType.DMA,
      ])
  def body(src_hbm, dst_hbm, ex_hbm, denr_hbm, hg_hbm, zagg_hbm, aggP_hbm,
           denr_v, src_c, dst_c, exc, rows, cvals, agg_sh, sem):
    cid, sid, wid = _wids()
    base = wid * EW
    pltpu.sync_copy(denr_hbm, denr_v)

    @pl.when(sid == 0)
    def _():
      pltpu.sync_copy(zagg_hbm, agg_sh)
    plsc.subcore_barrier()

    def chunk(g, carry):
      eb = base + g * C
      pltpu.sync_copy(src_hbm.at[pl.ds(eb, C)], src_c)
      pltpu.sync_copy(dst_hbm.at[pl.ds(eb, C)], dst_c)
      pltpu.sync_copy(ex_hbm.at[pl.ds(eb * HEADS, C * HEADS)], exc)
      pltpu.async_copy(hg_hbm.at[src_c], rows, sem).wait()
      for i in range(C // 16):
        e16 = lax.iota(i32, 16) + i * 16
        d16 = dst_c[pl.ds(i * 16, 16)]
        alphas = []
        for h in range(HEADS):
          h16 = jnp.full((16,), h, i32)
          exv = plsc.load_gather(exc, [e16 * HEADS + h16])
          drv = plsc.load_gather(denr_v, [d16 * HEADS + h16])
          alphas.append(exv * drv * 0.25)

        def colbody(ccol, carry2):
          c16 = jnp.zeros((16,), i32) + ccol
          acc = jnp.zeros((16,), f32)
          for h in range(HEADS):
            acc = acc + alphas[h] * plsc.load_gather(
                rows, [e16, c16 + h * HID])
          plsc.store_scatter(cvals, [e16, c16], acc)
          return carry2

        lax.fori_loop(0, HID, colbody, 0)
      pltpu.sync_copy(cvals, agg_sh.at[dst_c], add=True)
      return carry

    lax.fori_loop(0, nch, chunk, 0)
    plsc.subcore_barrier()

    @pl.when(sid == 0)
    def _():
      pltpu.sync_copy(agg_sh, aggP_hbm.at[cid])

  return body(src, dst, ex, denr, hg, zagg)


def _inj_score(dst, src, q, k, zden):
  CI = 160
  SB = 80
  EW = E_VI // NW
  nch = EW // CI

  @functools.partial(
      pl.kernel, mesh=_sc_mesh(),
      compiler_params=pltpu.CompilerParams(needs_layout_passes=False, use_tc_tiling_on_sc=False),
      out_type=(jax.ShapeDtypeStruct((E_VI,), f32),
                jax.ShapeDtypeStruct((NC, DEN_I), f32)),
      scratch_types=[
          pltpu.VMEM((CI,), i32), pltpu.VMEM((CI,), i32),
          pltpu.VMEM((CI,), i32), pltpu.VMEM((CI,), i32),
          pltpu.VMEM((CI, HID), f32), pltpu.VMEM((CI, HID), f32),
          pltpu.VMEM((CI, HID), f32), pltpu.VMEM((CI, HID), f32),
          pltpu.VMEM((CI,), f32),
          pltpu.VMEM((CI // SB, SB), i32),
          pltpu.VMEM_SHARED((DEN_I,), f32),
          pltpu.SemaphoreType.DMA, pltpu.SemaphoreType.DMA,
      ])
  def body(dst_hbm, src_hbm, q_hbm, k_hbm, zden_hbm, ex_hbm, denP_hbm,
           dst0, dst1, src0, src1, qr0, qr1, kr0, kr1,
           sbuf, ib2, den_sh, sem0, sem1):
    cid, sid, wid = _wids()
    base = wid * EW
    sets = ((dst0, src0, qr0, kr0, sem0),
            (dst1, src1, qr1, kr1, sem1))

    @pl.when(sid == 0)
    def _():
      pltpu.sync_copy(zden_hbm, den_sh)
    plsc.subcore_barrier()

    def prefetch(g, st):
      dstX, srcX, qrX, krX, semX = st
      eb = base + g * CI
      pltpu.sync_copy(dst_hbm.at[pl.ds(eb, CI)], dstX)
      pltpu.sync_copy(src_hbm.at[pl.ds(eb, CI)], srcX)
      for b in range(CI // SB):
        pltpu.async_copy(q_hbm.at[dstX.at[pl.ds(b * SB, SB)]],
                         qrX.at[pl.ds(b * SB, SB)], semX)
        pltpu.async_copy(k_hbm.at[srcX.at[pl.ds(b * SB, SB)]],
                         krX.at[pl.ds(b * SB, SB)], semX)

    def drain(st):
      dstX, srcX, qrX, krX, semX = st
      for b in range(CI // SB):
        pltpu.make_async_copy(q_hbm.at[dstX.at[pl.ds(b * SB, SB)]],
                              qrX.at[pl.ds(b * SB, SB)], semX).wait()
        pltpu.make_async_copy(k_hbm.at[srcX.at[pl.ds(b * SB, SB)]],
                              krX.at[pl.ds(b * SB, SB)], semX).wait()

    def compute(g, st):
      dstX, srcX, qrX, krX, semX = st
      eb = base + g * CI
      for i in range(CI // 16):
        e16 = lax.iota(i32, 16) + i * 16

        def colbody(ccol, acc):
          c16 = jnp.zeros((16,), i32) + ccol
          return acc + (plsc.load_gather(qrX, [e16, c16])
                        * plsc.load_gather(krX, [e16, c16]))

        acc = lax.fori_loop(0, HID, colbody, jnp.zeros((16,), f32))
        sbuf[pl.ds(i * 16, 16)] = jnp.exp(acc * (1.0 / SCALE))
        ib2[i // (SB // 16), pl.ds((i % (SB // 16)) * 16, 16)] = (
            dstX[pl.ds(i * 16, 16)])
      pltpu.sync_copy(sbuf, ex_hbm.at[pl.ds(eb, CI)])
      for b in range(CI // SB):
        pltpu.sync_copy(sbuf.at[pl.ds(b * SB, SB)],
                        den_sh.at[ib2.at[b]], add=True)

    prefetch(0, sets[0])

    def pairbody(g2, carry):
      gA = 2 * g2
      prefetch(gA + 1, sets[1])
      drain(sets[0])
      compute(gA, sets[0])
      prefetch(jnp.minimum(gA + 2, nch - 1), sets[0])
      drain(sets[1])
      compute(gA + 1, sets[1])
      return carry

    lax.fori_loop(0, nch // 2, pairbody, 0)
    drain(sets[0])
    if nch % 2 == 1:
      compute(nch - 1, sets[0])
    plsc.subcore_barrier()

    @pl.when(sid == 0)
    def _():
      pltpu.sync_copy(den_sh, denP_hbm.at[cid])

  return body(dst, src, q, k, zden)


def _alphaize(dst, ex, denr):
  EW = E_VI // NW
  nch = EW // C

  @functools.partial(
      pl.kernel, mesh=_sc_mesh(),
      compiler_params=pltpu.CompilerParams(needs_layout_passes=False, use_tc_tiling_on_sc=False),
      out_type=jax.ShapeDtypeStruct((E_VI,), f32),
      scratch_types=[
          pltpu.VMEM((DEN_I,), f32),
          pltpu.VMEM((C,), i32),
          pltpu.VMEM((C,), f32),
          pltpu.SemaphoreType.DMA,
      ])
  def body(dst_hbm, ex_hbm, denr_hbm, al_hbm, denr_v, dst_c, buf, sem):
    cid, sid, wid = _wids()
    base = wid * EW
    pltpu.sync_copy(denr_hbm, denr_v)

    def chunk(g, carry):
      eb = base + g * C
      pltpu.sync_copy(dst_hbm.at[pl.ds(eb, C)], dst_c)
      pltpu.sync_copy(ex_hbm.at[pl.ds(eb, C)], buf)
      for i in range(C // 16):
        sl = pl.ds(i * 16, 16)
        buf[sl] = buf[sl] * plsc.load_gather(denr_v, [dst_c[sl]])
      pltpu.sync_copy(buf, al_hbm.at[pl.ds(eb, C)])
      return carry

    lax.fori_loop(0, nch, chunk, 0)

  return body(dst, ex, denr)


def _inj_agg(dst, src, alpha, vals_cs, zmsg):
  CI = 160
  SB = 80
  NG = 16
  N_ENT = 10000
  EW = E_VI // NS
  nch = EW // CI

  @functools.partial(
      pl.kernel, mesh=_sc_mesh(),
      compiler_params=pltpu.CompilerParams(needs_layout_passes=False, use_tc_tiling_on_sc=False),
      out_type=jax.ShapeDtypeStruct((4 * N_VAR, NG), f32),
      scratch_types=[
          pltpu.VMEM((CI,), i32), pltpu.VMEM((CI,), i32),
          pltpu.VMEM((CI,), i32), pltpu.VMEM((CI,), i32),
          pltpu.VMEM((CI,), f32), pltpu.VMEM((CI,), f32),
          pltpu.VMEM((CI, NG), f32), pltpu.VMEM((CI, NG), f32),
          pltpu.VMEM((CI,), i32),
          pltpu.VMEM((CI // SB, SB), i32),
          pltpu.VMEM_SHARED((N_VAR, NG), f32),
          pltpu.SemaphoreType.DMA, pltpu.SemaphoreType.DMA,
      ])
  def body(dst_hbm, src_hbm, al_hbm, vals_hbm, zmsg_hbm, msg_hbm,
           dst0, dst1, sidx0, sidx1, al0, al1, rows0, rows1,
           srcb, ib2, msg_sh, sem0, sem1):
    cid, sid, wid = _wids()
    base = sid * EW
    sets = ((dst0, sidx0, al0, rows0, sem0),
            (dst1, sidx1, al1, rows1, sem1))

    def prefetch(g, st, sbase):
      dstX, sidxX, alX, rowsX, semX = st
      eb = base + g * CI
      pltpu.sync_copy(dst_hbm.at[pl.ds(eb, CI)], dstX)
      pltpu.sync_copy(src_hbm.at[pl.ds(eb, CI)], srcb)
      pltpu.sync_copy(al_hbm.at[pl.ds(eb, CI)], alX)
      for i in range(CI // 16):
        sl = pl.ds(i * 16, 16)
        sidxX[sl] = srcb[sl] + sbase
      for b in range(CI // SB):
        pltpu.async_copy(vals_hbm.at[sidxX.at[pl.ds(b * SB, SB)]],
                         rowsX.at[pl.ds(b * SB, SB)], semX)

    def drain(st):
      dstX, sidxX, alX, rowsX, semX = st
      for b in range(CI // SB):
        pltpu.make_async_copy(vals_hbm.at[sidxX.at[pl.ds(b * SB, SB)]],
                              rowsX.at[pl.ds(b * SB, SB)], semX).wait()

    def compute(st):
      dstX, sidxX, alX, rowsX, semX = st
      for i in range(CI // 16):
        sl = pl.ds(i * 16, 16)
        ib2[i // (SB // 16), pl.ds((i % (SB // 16)) * 16, 16)] = dstX[sl]

      def colbody(ccol, carry2):
        c16 = jnp.zeros((16,), i32) + ccol
        for i in range(CI // 16):
          e16 = lax.iota(i32, 16) + i * 16
          v16 = plsc.load_gather(rowsX, [e16, c16])
          plsc.store_scatter(rowsX, [e16, c16],
                             v16 * alX[pl.ds(i * 16, 16)])
        return carry2

      lax.fori_loop(0, NG, colbody, 0)
      for b in range(CI // SB):
        pltpu.sync_copy(rowsX.at[pl.ds(b * SB, SB)],
                        msg_sh.at[ib2.at[b]], add=True)

    for r in range(2):
      cg = lax.axis_index("c") * 2 + r
      sbase = cg * N_ENT

      @pl.when(sid == 0)
      def _():
        pltpu.sync_copy(zmsg_hbm, msg_sh)
      plsc.subcore_barrier()

      prefetch(0, sets[0], sbase)

      def pairbody(g2, carry):
        gA = 2 * g2
        prefetch(gA + 1, sets[1], sbase)
        drain(sets[0])
        compute(sets[0])
        prefetch(jnp.minimum(gA + 2, nch - 1), sets[0], sbase)
        drain(sets[1])
        compute(sets[1])
        return carry

      lax.fori_loop(0, nch // 2, pairbody, 0)
      drain(sets[0])
      plsc.subcore_barrier()

      @pl.when(sid == 0)
      def _():
        pltpu.sync_copy(msg_sh, msg_hbm.at[pl.ds(cg * N_VAR, N_VAR)])
      plsc.subcore_barrier()

  return body(dst, src, alpha, vals_cs, zmsg)


def kernel(x_employee, x_shift, x_variable, x_constraint, edge_same_day,
           edge_var_emp, edge_var_shift, W_proj_emp, b_proj_emp,
           W_proj_shift, b_proj_shift, W_proj_var, b_proj_var, W_proj_con,
           b_proj_con, W_gat, att_src, att_dst, b_gat, W_inj_emp, b_inj_emp,
           W_inj_shift, b_inj_shift, Wq_emp, Wk_emp, Wq_shift, Wk_shift,
           W_fuse, b_fuse):
  del x_constraint, W_proj_con, b_proj_con

  A_src = jnp.zeros((HEADS * HID, HEADS), f32)
  A_dst = jnp.zeros((HEADS * HID, HEADS), f32)
  for h in range(HEADS):
    A_src = A_src.at[h * HID:(h + 1) * HID, h].set(att_src[h])
    A_dst = A_dst.at[h * HID:(h + 1) * HID, h].set(att_dst[h])

  h_emp, k_emp, v_emp = _ent_pre(x_employee, W_proj_emp, b_proj_emp,
                                 Wk_emp, W_inj_emp, b_inj_emp)
  del h_emp
  h_shift0, hg, aS, aD = _shift_pre(x_shift, W_proj_shift, b_proj_shift,
                                    W_gat, A_src, A_dst)
  h_var, q_emp, q_shift = _var_pre(x_variable, W_proj_var, b_proj_var,
                                   Wq_emp, Wq_shift)

  src_sd = edge_same_day[0]
  dst_sd = edge_same_day[1]
  ex_g, denP_g = _gat_edge(src_sd, dst_sd, aS.reshape(-1), aD.reshape(-1),
                           jnp.zeros((DEN_G,), f32))
  denr_g = _denr(denP_g[0], denP_g[1], 1e-16)
  aggP = _gat_agg(src_sd, dst_sd, ex_g, denr_g, hg,
                  jnp.zeros((N_SHIFT, HID), f32))
  k_shift, v_shift = _post_gat(aggP[0], aggP[1], b_gat, h_shift0,
                               Wk_shift, W_inj_shift, b_inj_shift)

  zden_i = jnp.zeros((DEN_I,), f32)
  zmsg = jnp.zeros((N_VAR, 16), f32)

  def col_stack(v):
    return v.reshape(-1, 4, 16).transpose(1, 0, 2).reshape(-1, 16)

  def col_unstack(m):
    return m.reshape(4, N_VAR, 16).transpose(1, 0, 2).reshape(N_VAR, HID)

  d_ve = edge_var_emp[0]
  s_ve = edge_var_emp[1]
  ex_e, denP_e = _inj_score(d_ve, s_ve, q_emp, k_emp, zden_i)
  denr_e = _denr(denP_e[0], denP_e[1], 1e-9)
  al_e = _alphaize(d_ve, ex_e, denr_e)
  msg_e = col_unstack(_inj_agg(d_ve, s_ve, al_e, col_stack(v_emp), zmsg))

  d_vs = edge_var_shift[0]
  s_vs = edge_var_shift[1]
  ex_s, denP_s = _inj_score(d_vs, s_vs, q_shift, k_shift, zden_i)
  denr_s = _denr(denP_s[0], denP_s[1], 1e-9)
  al_s = _alphaize(d_vs, ex_s, denr_s)
  msg_s = col_unstack(_inj_agg(d_vs, s_vs, al_s, col_stack(v_shift), zmsg))

  W1 = W_fuse[:HID]
  W2 = W_fuse[HID:2 * HID]
  W3 = W_fuse[2 * HID:]
  return _fuse(h_var, msg_e, msg_s, W1, W2, W3, b_fuse)


# speedup vs baseline: 8.8491x; 1.0193x over previous
import functools

import jax
import jax.numpy as jnp
from jax import lax
from jax.experimental import pallas as pl
from jax.experimental.pallas import tpu as pltpu
from jax.experimental.pallas import tpu_sc as plsc

HID = 64
HEADS = 4
SCALE = 8.0
NC = 2
NS = 16
NW = NC * NS
C = 80

N_VAR = 100000
N_SHIFT = 10000
N_EMP = 10000
E_SD = 320000
E_VI = 640000
DEN_G = 40960
DEN_I = 100352
RNG = 25000

f32 = jnp.float32
i32 = jnp.int32


def _ent_pre(x, W, b, Wk, Wv, bv):
  N = x.shape[0]
  BR = 1000
  def body(x_ref, W_ref, b_ref, Wk_ref, Wv_ref, bv_ref, h_ref, k_ref, v_ref):
    h = jnp.maximum(x_ref[...] @ W_ref[...] + b_ref[...], 0.0)
    h_ref[...] = h
    k_ref[...] = h @ Wk_ref[...]
    v_ref[...] = h @ Wv_ref[...] + bv_ref[...]
  K = x.shape[1]
  return pl.pallas_call(
      body,
      grid=(N // BR,),
      in_specs=[
          pl.BlockSpec((BR, K), lambda i: (i, 0)),
          pl.BlockSpec((K, HID), lambda i: (0, 0)),
          pl.BlockSpec((1, HID), lambda i: (0, 0)),
          pl.BlockSpec((HID, HID), lambda i: (0, 0)),
          pl.BlockSpec((HID, HID), lambda i: (0, 0)),
          pl.BlockSpec((1, HID), lambda i: (0, 0)),
      ],
      out_specs=[
          pl.BlockSpec((BR, HID), lambda i: (i, 0)),
          pl.BlockSpec((BR, HID), lambda i: (i, 0)),
          pl.BlockSpec((BR, HID), lambda i: (i, 0)),
      ],
      out_shape=[jax.ShapeDtypeStruct((N, HID), f32)] * 3,
  )(x, W, b.reshape(1, HID), Wk, Wv, bv.reshape(1, HID))


def _shift_pre(x, W, b, W_gat, A_src, A_dst):
  N = x.shape[0]
  BR = 1000
  K = x.shape[1]
  def body(x_ref, W_ref, b_ref, Wg_ref, As_ref, Ad_ref,
           h_ref, hg_ref, as_ref, ad_ref):
    h = jnp.maximum(x_ref[...] @ W_ref[...] + b_ref[...], 0.0)
    hg = h @ Wg_ref[...]
    h_ref[...] = h
    hg_ref[...] = hg
    as_ref[...] = hg @ As_ref[...]
    ad_ref[...] = hg @ Ad_ref[...]
  return pl.pallas_call(
      body,
      grid=(N // BR,),
      in_specs=[
          pl.BlockSpec((BR, K), lambda i: (i, 0)),
          pl.BlockSpec((K, HID), lambda i: (0, 0)),
          pl.BlockSpec((1, HID), lambda i: (0, 0)),
          pl.BlockSpec((HID, HEADS * HID), lambda i: (0, 0)),
          pl.BlockSpec((HEADS * HID, HEADS), lambda i: (0, 0)),
          pl.BlockSpec((HEADS * HID, HEADS), lambda i: (0, 0)),
      ],
      out_specs=[
          pl.BlockSpec((BR, HID), lambda i: (i, 0)),
          pl.BlockSpec((BR, HEADS * HID), lambda i: (i, 0)),
          pl.BlockSpec((BR, HEADS), lambda i: (i, 0)),
          pl.BlockSpec((BR, HEADS), lambda i: (i, 0)),
      ],
      out_shape=[
          jax.ShapeDtypeStruct((N, HID), f32),
          jax.ShapeDtypeStruct((N, HEADS * HID), f32),
          jax.ShapeDtypeStruct((N, HEADS), f32),
          jax.ShapeDtypeStruct((N, HEADS), f32),
      ],
  )(x, W, b.reshape(1, HID), W_gat, A_src, A_dst)


def _var_pre(x, W, b, Wq1, Wq2):
  N = x.shape[0]
  BR = 1000
  K = x.shape[1]
  def body(x_ref, W_ref, b_ref, W1_ref, W2_ref, h_ref, q1_ref, q2_ref):
    h = jnp.maximum(x_ref[...] @ W_ref[...] + b_ref[...], 0.0)
    h_ref[...] = h
    q1_ref[...] = h @ W1_ref[...]
    q2_ref[...] = h @ W2_ref[...]
  return pl.pallas_call(
      body,
      grid=(N // BR,),
      in_specs=[
          pl.BlockSpec((BR, K), lambda i: (i, 0)),
          pl.BlockSpec((K, HID), lambda i: (0, 0)),
          pl.BlockSpec((1, HID), lambda i: (0, 0)),
          pl.BlockSpec((HID, HID), lambda i: (0, 0)),
          pl.BlockSpec((HID, HID), lambda i: (0, 0)),
      ],
      out_specs=[
          pl.BlockSpec((BR, HID), lambda i: (i, 0)),
          pl.BlockSpec((BR, HID), lambda i: (i, 0)),
          pl.BlockSpec((BR, HID), lambda i: (i, 0)),
      ],
      out_shape=[jax.ShapeDtypeStruct((N, HID), f32)] * 3,
  )(x, W, b.reshape(1, HID), Wq1, Wq2)


def _denr(d0, d1, clip):
  M = d0.shape[0]
  rows = M // 128
  def body(a_ref, b_ref, o_ref):
    o_ref[...] = 1.0 / jnp.maximum(a_ref[...] + b_ref[...], clip)
  out = pl.pallas_call(
      body,
      out_shape=jax.ShapeDtypeStruct((rows, 128), f32),
  )(d0.reshape(rows, 128), d1.reshape(rows, 128))
  return out.reshape(M)


def _post_gat(agg0, agg1, b_gat, h0, Wk, Wv, bv):
  N = h0.shape[0]
  BR = 1000
  def body(a0_ref, a1_ref, bg_ref, h0_ref, Wk_ref, Wv_ref, bv_ref,
           k_ref, v_ref):
    g = a0_ref[...] + a1_ref[...] + bg_ref[...]
    hs = jnp.maximum(g, 0.0) + h0_ref[...]
    k_ref[...] = hs @ Wk_ref[...]
    v_ref[...] = hs @ Wv_ref[...] + bv_ref[...]
  return pl.pallas_call(
      body,
      grid=(N // BR,),
      in_specs=[
          pl.BlockSpec((BR, HID), lambda i: (i, 0)),
          pl.BlockSpec((BR, HID), lambda i: (i, 0)),
          pl.BlockSpec((1, HID), lambda i: (0, 0)),
          pl.BlockSpec((BR, HID), lambda i: (i, 0)),
          pl.BlockSpec((HID, HID), lambda i: (0, 0)),
          pl.BlockSpec((HID, HID), lambda i: (0, 0)),
          pl.BlockSpec((1, HID), lambda i: (0, 0)),
      ],
      out_specs=[
          pl.BlockSpec((BR, HID), lambda i: (i, 0)),
          pl.BlockSpec((BR, HID), lambda i: (i, 0)),
      ],
      out_shape=[jax.ShapeDtypeStruct((N, HID), f32)] * 2,
  )(agg0, agg1, b_gat.reshape(1, HID), h0, Wk, Wv, bv.reshape(1, HID))


def _fuse(hv, mE, mS, W1, W2, W3, b):
  N = hv.shape[0]
  BR = 1000
  def body(hv_ref, mE_ref, mS_ref, W1_ref, W2_ref, W3_ref, b_ref, o_ref):
    z = (hv_ref[...] @ W1_ref[...] + mE_ref[...] @ W2_ref[...]
         + mS_ref[...] @ W3_ref[...] + b_ref[...])
    z = jnp.maximum(z, 0.0)
    o_ref[...] = jnp.sqrt(jnp.sum(z * z, axis=1, keepdims=True))
  out = pl.pallas_call(
      body,
      grid=(N // BR,),
      in_specs=[
          pl.BlockSpec((BR, HID), lambda i: (i, 0)),
          pl.BlockSpec((BR, HID), lambda i: (i, 0)),
          pl.BlockSpec((BR, HID), lambda i: (i, 0)),
          pl.BlockSpec((HID, HID), lambda i: (0, 0)),
          pl.BlockSpec((HID, HID), lambda i: (0, 0)),
          pl.BlockSpec((HID, HID), lambda i: (0, 0)),
          pl.BlockSpec((1, HID), lambda i: (0, 0)),
      ],
      out_specs=pl.BlockSpec((BR, 1), lambda i: (i, 0)),
      out_shape=jax.ShapeDtypeStruct((N, 1), f32),
  )(hv, mE, mS, W1, W2, W3, b.reshape(1, HID))
  return out.reshape(N)


def _sc_mesh():
  return plsc.VectorSubcoreMesh(core_axis_name="c", subcore_axis_name="s")


def _wids():
  cid = lax.axis_index("c")
  sid = lax.axis_index("s")
  return cid, sid, sid * NC + cid


def _gat_edge(src, dst, aS, aD, zden):
  EW = E_SD // NW
  nch = EW // C

  @functools.partial(
      pl.kernel, mesh=_sc_mesh(),
      compiler_params=pltpu.CompilerParams(needs_layout_passes=False, use_tc_tiling_on_sc=False),
      out_type=(jax.ShapeDtypeStruct((E_SD * HEADS,), f32),
                jax.ShapeDtypeStruct((NC, DEN_G), f32)),
      scratch_types=[
          pltpu.VMEM((N_SHIFT * HEADS,), f32),
          pltpu.VMEM((N_SHIFT * HEADS,), f32),
          pltpu.VMEM((C,), i32),
          pltpu.VMEM((C,), i32),
          pltpu.VMEM((C * HEADS,), f32),
          pltpu.VMEM((HEADS, C), f32),
          pltpu.VMEM((HEADS, C), i32),
          pltpu.VMEM_SHARED((DEN_G,), f32),
          pltpu.SemaphoreType.DMA,
      ])
  def body(src_hbm, dst_hbm, aS_hbm, aD_hbm, zden_hbm, ex_hbm, denP_hbm,
           aS_v, aD_v, src_c, dst_c, exc, exh, ibuf, den_sh, sem):
    cid, sid, wid = _wids()
    base = wid * EW
    pltpu.sync_copy(aS_hbm, aS_v)
    pltpu.sync_copy(aD_hbm, aD_v)

    @pl.when(sid == 0)
    def _():
      pltpu.sync_copy(zden_hbm, den_sh)
    plsc.subcore_barrier()

    def chunk(g, carry):
      eb = base + g * C
      pltpu.sync_copy(src_hbm.at[pl.ds(eb, C)], src_c)
      pltpu.sync_copy(dst_hbm.at[pl.ds(eb, C)], dst_c)
      for i in range(C // 16):
        loc16 = lax.iota(i32, 16) + i * 16
        s16 = src_c[pl.ds(i * 16, 16)]
        d16 = dst_c[pl.ds(i * 16, 16)]
        for h in range(HEADS):
          h16 = jnp.full((16,), h, i32)
          e16 = (plsc.load_gather(aS_v, [s16 * HEADS + h16])
                 + plsc.load_gather(aD_v, [d16 * HEADS + h16]))
          e16 = jnp.where(e16 >= 0.0, e16, 0.2 * e16)
          ex16 = jnp.exp(e16)
          plsc.store_scatter(exc, [loc16 * HEADS + h16], ex16)
          exh[h, pl.ds(i * 16, 16)] = ex16
          ibuf[h, pl.ds(i * 16, 16)] = d16 * HEADS + h16
      pltpu.sync_copy(exc, ex_hbm.at[pl.ds(eb * HEADS, C * HEADS)])
      for h in range(HEADS):
        pltpu.sync_copy(exh.at[h], den_sh.at[ibuf.at[h]], add=True)
      return carry

    lax.fori_loop(0, nch, chunk, 0)
    plsc.subcore_barrier()

    @pl.when(sid == 0)
    def _():
      pltpu.sync_copy(den_sh, denP_hbm.at[cid])

  return body(src, dst, aS, aD, zden)


def _gat_agg(src, dst, ex, denr, hg, zagg):
  EW = E_SD // NW
  nch = EW // C

  @functools.partial(
      pl.kernel, mesh=_sc_mesh(),
      compiler_params=pltpu.CompilerParams(needs_layout_passes=False, use_tc_tiling_on_sc=False),
      out_type=jax.ShapeDtypeStruct((NC, N_SHIFT, HID), f32),
      scratch_types=[
          pltpu.VMEM((DEN_G,), f32),
          pltpu.VMEM((C,), i32), pltpu.VMEM((C,), i32),
          pltpu.VMEM((C,), i32), pltpu.VMEM((C,), i32),
          pltpu.VMEM((C * HEADS,), f32), pltpu.VMEM((C * HEADS,), f32),
          pltpu.VMEM((C, HEADS * HID), f32), pltpu.VMEM((C, HEADS * HID), f32),
          pltpu.VMEM((C, HID), f32),
          pltpu.VMEM_SHARED((N_SHIFT, HID), f32),
          pltpu.SemaphoreType.DMA, pltpu.SemaphoreType.DMA,
      ])
  def body(src_hbm, dst_hbm, ex_hbm, denr_hbm, hg_hbm, zagg_hbm, aggP_hbm,
           denr_v, src0, src1, dst0, dst1, exc0, exc1, rows0, rows1,
           cvals, agg_sh, sem0, sem1):
    cid, sid, wid = _wids()
    base = wid * EW
    sets = ((src0, dst0, exc0, rows0, sem0),
            (src1, dst1, exc1, rows1, sem1))
    pltpu.sync_copy(denr_hbm, denr_v)

    @pl.when(sid == 0)
    def _():
      pltpu.sync_copy(zagg_hbm, agg_sh)
    plsc.subcore_barrier()

    def prefetch(g, st):
      srcX, dstX, excX, rowsX, semX = st
      eb = base + g * C
      pltpu.sync_copy(src_hbm.at[pl.ds(eb, C)], srcX)
      pltpu.sync_copy(dst_hbm.at[pl.ds(eb, C)], dstX)
      pltpu.sync_copy(ex_hbm.at[pl.ds(eb * HEADS, C * HEADS)], excX)
      pltpu.async_copy(hg_hbm.at[srcX], rowsX, semX)

    def drain(st):
      srcX, dstX, excX, rowsX, semX = st
      pltpu.make_async_copy(hg_hbm.at[srcX], rowsX, semX).wait()

    def compute(st):
      srcX, dstX, excX, rowsX, semX = st
      for i in range(C // 16):
        e16 = lax.iota(i32, 16) + i * 16
        d16 = dstX[pl.ds(i * 16, 16)]
        alphas = []
        for h in range(HEADS):
          h16 = jnp.full((16,), h, i32)
          exv = plsc.load_gather(excX, [e16 * HEADS + h16])
          drv = plsc.load_gather(denr_v, [d16 * HEADS + h16])
          alphas.append(exv * drv * 0.25)

        def colbody(ccol, carry2):
          c16 = jnp.zeros((16,), i32) + ccol
          acc = jnp.zeros((16,), f32)
          for h in range(HEADS):
            acc = acc + alphas[h] * plsc.load_gather(
                rowsX, [e16, c16 + h * HID])
          plsc.store_scatter(cvals, [e16, c16], acc)
          return carry2

        lax.fori_loop(0, HID, colbody, 0)
      pltpu.sync_copy(cvals, agg_sh.at[dstX], add=True)

    prefetch(0, sets[0])

    def pairbody(g2, carry):
      gA = 2 * g2
      prefetch(gA + 1, sets[1])
      drain(sets[0])
      compute(sets[0])
      prefetch(jnp.minimum(gA + 2, nch - 1), sets[0])
      drain(sets[1])
      compute(sets[1])
      return carry

    lax.fori_loop(0, nch // 2, pairbody, 0)
    drain(sets[0])
    if nch % 2 == 1:
      compute(sets[0])
    plsc.subcore_barrier()

    @pl.when(sid == 0)
    def _():
      pltpu.sync_copy(agg_sh, aggP_hbm.at[cid])

  return body(src, dst, ex, denr, hg, zagg)


def _inj_score(dst, src, q, k, zden):
  CI = 160
  SB = 80
  EW = E_VI // NW
  nch = EW // CI

  @functools.partial(
      pl.kernel, mesh=_sc_mesh(),
      compiler_params=pltpu.CompilerParams(needs_layout_passes=False, use_tc_tiling_on_sc=False),
      out_type=(jax.ShapeDtypeStruct((E_VI,), f32),
                jax.ShapeDtypeStruct((NC, DEN_I), f32)),
      scratch_types=[
          pltpu.VMEM((CI,), i32), pltpu.VMEM((CI,), i32),
          pltpu.VMEM((CI,), i32), pltpu.VMEM((CI,), i32),
          pltpu.VMEM((CI, HID), f32), pltpu.VMEM((CI, HID), f32),
          pltpu.VMEM((CI, HID), f32), pltpu.VMEM((CI, HID), f32),
          pltpu.VMEM((CI,), f32),
          pltpu.VMEM((CI // SB, SB), i32),
          pltpu.VMEM_SHARED((DEN_I,), f32),
          pltpu.SemaphoreType.DMA, pltpu.SemaphoreType.DMA,
      ])
  def body(dst_hbm, src_hbm, q_hbm, k_hbm, zden_hbm, ex_hbm, denP_hbm,
           dst0, dst1, src0, src1, qr0, qr1, kr0, kr1,
           sbuf, ib2, den_sh, sem0, sem1):
    cid, sid, wid = _wids()
    base = wid * EW
    sets = ((dst0, src0, qr0, kr0, sem0),
            (dst1, src1, qr1, kr1, sem1))

    @pl.when(sid == 0)
    def _():
      pltpu.sync_copy(zden_hbm, den_sh)
    plsc.subcore_barrier()

    def prefetch(g, st):
      dstX, srcX, qrX, krX, semX = st
      eb = base + g * CI
      pltpu.sync_copy(dst_hbm.at[pl.ds(eb, CI)], dstX)
      pltpu.sync_copy(src_hbm.at[pl.ds(eb, CI)], srcX)
      for b in range(CI // SB):
        pltpu.async_copy(q_hbm.at[dstX.at[pl.ds(b * SB, SB)]],
                         qrX.at[pl.ds(b * SB, SB)], semX)
        pltpu.async_copy(k_hbm.at[srcX.at[pl.ds(b * SB, SB)]],
                         krX.at[pl.ds(b * SB, SB)], semX)

    def drain(st):
      dstX, srcX, qrX, krX, semX = st
      for b in range(CI // SB):
        pltpu.make_async_copy(q_hbm.at[dstX.at[pl.ds(b * SB, SB)]],
                              qrX.at[pl.ds(b * SB, SB)], semX).wait()
        pltpu.make_async_copy(k_hbm.at[srcX.at[pl.ds(b * SB, SB)]],
                              krX.at[pl.ds(b * SB, SB)], semX).wait()

    def compute(g, st):
      dstX, srcX, qrX, krX, semX = st
      eb = base + g * CI
      for i in range(CI // 16):
        e16 = lax.iota(i32, 16) + i * 16

        def colbody(ccol, acc):
          c16 = jnp.zeros((16,), i32) + ccol
          return acc + (plsc.load_gather(qrX, [e16, c16])
                        * plsc.load_gather(krX, [e16, c16]))

        acc = lax.fori_loop(0, HID, colbody, jnp.zeros((16,), f32))
        sbuf[pl.ds(i * 16, 16)] = jnp.exp(acc * (1.0 / SCALE))
        ib2[i // (SB // 16), pl.ds((i % (SB // 16)) * 16, 16)] = (
            dstX[pl.ds(i * 16, 16)])
      pltpu.sync_copy(sbuf, ex_hbm.at[pl.ds(eb, CI)])
      for b in range(CI // SB):
        pltpu.sync_copy(sbuf.at[pl.ds(b * SB, SB)],
                        den_sh.at[ib2.at[b]], add=True)

    prefetch(0, sets[0])

    def pairbody(g2, carry):
      gA = 2 * g2
      prefetch(gA + 1, sets[1])
      drain(sets[0])
      compute(gA, sets[0])
      prefetch(jnp.minimum(gA + 2, nch - 1), sets[0])
      drain(sets[1])
      compute(gA + 1, sets[1])
      return carry

    lax.fori_loop(0, nch // 2, pairbody, 0)
    drain(sets[0])
    if nch % 2 == 1:
      compute(nch - 1, sets[0])
    plsc.subcore_barrier()

    @pl.when(sid == 0)
    def _():
      pltpu.sync_copy(den_sh, denP_hbm.at[cid])

  return body(dst, src, q, k, zden)


def _alphaize(dst, ex, denr):
  EW = E_VI // NW
  nch = EW // C

  @functools.partial(
      pl.kernel, mesh=_sc_mesh(),
      compiler_params=pltpu.CompilerParams(needs_layout_passes=False, use_tc_tiling_on_sc=False),
      out_type=jax.ShapeDtypeStruct((E_VI,), f32),
      scratch_types=[
          pltpu.VMEM((DEN_I,), f32),
          pltpu.VMEM((C,), i32),
          pltpu.VMEM((C,), f32),
          pltpu.SemaphoreType.DMA,
      ])
  def body(dst_hbm, ex_hbm, denr_hbm, al_hbm, denr_v, dst_c, buf, sem):
    cid, sid, wid = _wids()
    base = wid * EW
    pltpu.sync_copy(denr_hbm, denr_v)

    def chunk(g, carry):
      eb = base + g * C
      pltpu.sync_copy(dst_hbm.at[pl.ds(eb, C)], dst_c)
      pltpu.sync_copy(ex_hbm.at[pl.ds(eb, C)], buf)
      for i in range(C // 16):
        sl = pl.ds(i * 16, 16)
        buf[sl] = buf[sl] * plsc.load_gather(denr_v, [dst_c[sl]])
      pltpu.sync_copy(buf, al_hbm.at[pl.ds(eb, C)])
      return carry

    lax.fori_loop(0, nch, chunk, 0)

  return body(dst, ex, denr)


def _inj_agg(dst, src, alpha, vals_cs, zmsg):
  CI = 160
  SB = 80
  NG = 16
  N_ENT = 10000
  EW = E_VI // NS
  nch = EW // CI

  @functools.partial(
      pl.kernel, mesh=_sc_mesh(),
      compiler_params=pltpu.CompilerParams(needs_layout_passes=False, use_tc_tiling_on_sc=False),
      out_type=jax.ShapeDtypeStruct((4 * N_VAR, NG), f32),
      scratch_types=[
          pltpu.VMEM((CI,), i32), pltpu.VMEM((CI,), i32),
          pltpu.VMEM((CI,), i32), pltpu.VMEM((CI,), i32),
          pltpu.VMEM((CI,), f32), pltpu.VMEM((CI,), f32),
          pltpu.VMEM((CI, NG), f32), pltpu.VMEM((CI, NG), f32),
          pltpu.VMEM((CI,), i32),
          pltpu.VMEM((CI // SB, SB), i32),
          pltpu.VMEM_SHARED((N_VAR, NG), f32),
          pltpu.SemaphoreType.DMA, pltpu.SemaphoreType.DMA,
      ])
  def body(dst_hbm, src_hbm, al_hbm, vals_hbm, zmsg_hbm, msg_hbm,
           dst0, dst1, sidx0, sidx1, al0, al1, rows0, rows1,
           srcb, ib2, msg_sh, sem0, sem1):
    cid, sid, wid = _wids()
    base = sid * EW
    sets = ((dst0, sidx0, al0, rows0, sem0),
            (dst1, sidx1, al1, rows1, sem1))

    def prefetch(g, st, sbase):
      dstX, sidxX, alX, rowsX, semX = st
      eb = base + g * CI
      pltpu.sync_copy(dst_hbm.at[pl.ds(eb, CI)], dstX)
      pltpu.sync_copy(src_hbm.at[pl.ds(eb, CI)], srcb)
      pltpu.sync_copy(al_hbm.at[pl.ds(eb, CI)], alX)
      for i in range(CI // 16):
        sl = pl.ds(i * 16, 16)
        sidxX[sl] = srcb[sl] + sbase
      for b in range(CI // SB):
        pltpu.async_copy(vals_hbm.at[sidxX.at[pl.ds(b * SB, SB)]],
                         rowsX.at[pl.ds(b * SB, SB)], semX)

    def drain(st):
      dstX, sidxX, alX, rowsX, semX = st
      for b in range(CI // SB):
        pltpu.make_async_copy(vals_hbm.at[sidxX.at[pl.ds(b * SB, SB)]],
                              rowsX.at[pl.ds(b * SB, SB)], semX).wait()

    def compute(st):
      dstX, sidxX, alX, rowsX, semX = st
      for i in range(CI // 16):
        sl = pl.ds(i * 16, 16)
        ib2[i // (SB // 16), pl.ds((i % (SB // 16)) * 16, 16)] = dstX[sl]

      def colbody(ccol, carry2):
        c16 = jnp.zeros((16,), i32) + ccol
        for i in range(CI // 16):
          e16 = lax.iota(i32, 16) + i * 16
          v16 = plsc.load_gather(rowsX, [e16, c16])
          plsc.store_scatter(rowsX, [e16, c16],
                             v16 * alX[pl.ds(i * 16, 16)])
        return carry2

      lax.fori_loop(0, NG, colbody, 0)
      for b in range(CI // SB):
        pltpu.sync_copy(rowsX.at[pl.ds(b * SB, SB)],
                        msg_sh.at[ib2.at[b]], add=True)

    for r in range(2):
      cg = lax.axis_index("c") * 2 + r
      sbase = cg * N_ENT

      @pl.when(sid == 0)
      def _():
        pltpu.sync_copy(zmsg_hbm, msg_sh)
      plsc.subcore_barrier()

      prefetch(0, sets[0], sbase)

      def pairbody(g2, carry):
        gA = 2 * g2
        prefetch(gA + 1, sets[1], sbase)
        drain(sets[0])
        compute(sets[0])
        prefetch(jnp.minimum(gA + 2, nch - 1), sets[0], sbase)
        drain(sets[1])
        compute(sets[1])
        return carry

      lax.fori_loop(0, nch // 2, pairbody, 0)
      drain(sets[0])
      plsc.subcore_barrier()

      @pl.when(sid == 0)
      def _():
        pltpu.sync_copy(msg_sh, msg_hbm.at[pl.ds(cg * N_VAR, N_VAR)])
      plsc.subcore_barrier()

  return body(dst, src, alpha, vals_cs, zmsg)


def kernel(x_employee, x_shift, x_variable, x_constraint, edge_same_day,
           edge_var_emp, edge_var_shift, W_proj_emp, b_proj_emp,
           W_proj_shift, b_proj_shift, W_proj_var, b_proj_var, W_proj_con,
           b_proj_con, W_gat, att_src, att_dst, b_gat, W_inj_emp, b_inj_emp,
           W_inj_shift, b_inj_shift, Wq_emp, Wk_emp, Wq_shift, Wk_shift,
           W_fuse, b_fuse):
  del x_constraint, W_proj_con, b_proj_con

  A_src = jnp.zeros((HEADS * HID, HEADS), f32)
  A_dst = jnp.zeros((HEADS * HID, HEADS), f32)
  for h in range(HEADS):
    A_src = A_src.at[h * HID:(h + 1) * HID, h].set(att_src[h])
    A_dst = A_dst.at[h * HID:(h + 1) * HID, h].set(att_dst[h])

  h_emp, k_emp, v_emp = _ent_pre(x_employee, W_proj_emp, b_proj_emp,
                                 Wk_emp, W_inj_emp, b_inj_emp)
  del h_emp
  h_shift0, hg, aS, aD = _shift_pre(x_shift, W_proj_shift, b_proj_shift,
                                    W_gat, A_src, A_dst)
  h_var, q_emp, q_shift = _var_pre(x_variable, W_proj_var, b_proj_var,
                                   Wq_emp, Wq_shift)

  src_sd = edge_same_day[0]
  dst_sd = edge_same_day[1]
  ex_g, denP_g = _gat_edge(src_sd, dst_sd, aS.reshape(-1), aD.reshape(-1),
                           jnp.zeros((DEN_G,), f32))
  denr_g = _denr(denP_g[0], denP_g[1], 1e-16)
  aggP = _gat_agg(src_sd, dst_sd, ex_g, denr_g, hg,
                  jnp.zeros((N_SHIFT, HID), f32))
  k_shift, v_shift = _post_gat(aggP[0], aggP[1], b_gat, h_shift0,
                               Wk_shift, W_inj_shift, b_inj_shift)

  zden_i = jnp.zeros((DEN_I,), f32)
  zmsg = jnp.zeros((N_VAR, 16), f32)

  def col_stack(v):
    return v.reshape(-1, 4, 16).transpose(1, 0, 2).reshape(-1, 16)

  def col_unstack(m):
    return m.reshape(4, N_VAR, 16).transpose(1, 0, 2).reshape(N_VAR, HID)

  d_ve = edge_var_emp[0]
  s_ve = edge_var_emp[1]
  ex_e, denP_e = _inj_score(d_ve, s_ve, q_emp, k_emp, zden_i)
  denr_e = _denr(denP_e[0], denP_e[1], 1e-9)
  al_e = _alphaize(d_ve, ex_e, denr_e)
  msg_e = col_unstack(_inj_agg(d_ve, s_ve, al_e, col_stack(v_emp), zmsg))

  d_vs = edge_var_shift[0]
  s_vs = edge_var_shift[1]
  ex_s, denP_s = _inj_score(d_vs, s_vs, q_shift, k_shift, zden_i)
  denr_s = _denr(denP_s[0], denP_s[1], 1e-9)
  al_s = _alphaize(d_vs, ex_s, denr_s)
  msg_s = col_unstack(_inj_agg(d_vs, s_vs, al_s, col_stack(v_shift), zmsg))

  W1 = W_fuse[:HID]
  W2 = W_fuse[HID:2 * HID]
  W3 = W_fuse[2 * HID:]
  return _fuse(h_var, msg_e, msg_s, W1, W2, W3, b_fuse)
